# baseline (speedup 1.0000x reference)
import functools

import jax
import jax.numpy as jnp
from jax import lax
from jax.experimental import pallas as pl
from jax.experimental.pallas import tpu as pltpu
from jax.experimental.pallas import tpu_sc as plsc

_N = 10000
_E = 320000
_D = 128
_H = 256
_OUT = 128

_NT = 16
_NC = 2
_C = 80
_K0 = 10
_C1 = 40
_K1 = 8
_KD = 10
_SBR = 25
_HALFW = 72
_DEGW = 16
_NP = 10240
_NPT = _NP // _NT

_mesh = plsc.VectorSubcoreMesh(core_axis_name="c", subcore_axis_name="s")
_f32 = jnp.float32


@functools.partial(
    pl.kernel,
    out_type=[jax.ShapeDtypeStruct((_NP, _DEGW), _f32),
              jax.ShapeDtypeStruct((_NP, _DEGW), _f32)],
    mesh=_mesh,
    scratch_types=[pltpu.VMEM((_C, _DEGW), _f32),
                   pltpu.VMEM((_KD, _C), jnp.int32),
                   pltpu.VMEM_SHARED((_NP, _DEGW), _f32),
                   pltpu.SemaphoreType.DMA],
    compiler_params=pltpu.CompilerParams(use_tc_tiling_on_sc=False),
)
def _deg_kernel(send2d_hbm, recv2d_hbm, ones_hbm, zeros_hbm, ds_hbm, dr_hbm,
                ones_v, idx_v, acc, sem):
    c = lax.axis_index("c")
    s = lax.axis_index("s")
    pltpu.sync_copy(zeros_hbm, acc.at[pl.ds(s * _NPT, _NPT)])
    pltpu.sync_copy(ones_hbm, ones_v)
    plsc.subcore_barrier()

    rpt = (_E // _C) // _NT

    def hist(idx_hbm, out_hbm):
        def block(b, carry):
            r0 = s * rpt + b * _KD
            pltpu.sync_copy(idx_hbm.at[pl.ds(r0, _KD)], idx_v)
            adds = [pltpu.async_copy(ones_v, acc.at[idx_v.at[j]], sem, add=True)
                    for j in range(_KD)]
            for t in adds:
                t.wait()
            return carry
        lax.fori_loop(0, rpt // _KD, block, 0)
        plsc.subcore_barrier()
        pltpu.sync_copy(acc.at[pl.ds(s * _NPT, _NPT)],
                        out_hbm.at[pl.ds(s * _NPT, _NPT)])

    @pl.when(c == 0)
    def _():
        hist(send2d_hbm, ds_hbm)

    @pl.when(c == 1)
    def _():
        hist(recv2d_hbm, dr_hbm)


def _seg_body(table_hbm, out_hbm, send2d, recv2d, zeros_hbm,
              sidx, ridx, rows_v, acc, gsems, ssems, s, k, nsb, lag, csz):
    pltpu.sync_copy(zeros_hbm, acc.at[pl.ds(s * _NPT, _NPT)])
    plsc.subcore_barrier()
    row_base = s * (nsb * _SBR)

    def superblock(sb, carry):
        r0 = row_base + sb * _SBR
        pltpu.sync_copy(send2d.at[pl.ds(r0, _SBR)], sidx)
        pltpu.sync_copy(recv2d.at[pl.ds(r0, _SBR)], ridx)

        gdesc = [None] * _SBR
        sdesc = [None] * _SBR

        def fire_scatter(t):
            slot = t % k
            gdesc[t].wait()
            sdesc[t] = pltpu.async_copy(rows_v.at[pl.ds(slot * csz, csz)],
                                        acc.at[ridx.at[t]], ssems.at[slot],
                                        add=True)

        for j in range(_SBR):
            slot = j % k
            if j >= k:
                sdesc[j - k].wait()
            gdesc[j] = pltpu.async_copy(table_hbm.at[sidx.at[j]],
                                        rows_v.at[pl.ds(slot * csz, csz)],
                                        gsems.at[slot])
            if j >= lag:
                fire_scatter(j - lag)
        for t in range(_SBR - lag, _SBR):
            fire_scatter(t)
        for t in range(_SBR - k, _SBR):
            sdesc[t].wait()
        return carry

    lax.fori_loop(0, nsb, superblock, 0)
    plsc.subcore_barrier()
    pltpu.sync_copy(acc.at[pl.ds(s * _NPT, _NPT)],
                    out_hbm.at[pl.ds(s * _NPT, _NPT)])


@functools.partial(
    pl.kernel,
    out_type=[jax.ShapeDtypeStruct((_NP, _HALFW), _f32),
              jax.ShapeDtypeStruct((_NP, _HALFW), _f32)],
    mesh=_mesh,
    scratch_types=[pltpu.VMEM((_SBR, _C), jnp.int32),
                   pltpu.VMEM((_SBR, _C), jnp.int32),
                   pltpu.VMEM((_K0 * _C, _HALFW), _f32),
                   pltpu.VMEM_SHARED((_NP, _HALFW), _f32),
                   pltpu.SemaphoreType.DMA((_K0,)),
                   pltpu.SemaphoreType.DMA((_K0,))],
    compiler_params=pltpu.CompilerParams(use_tc_tiling_on_sc=False),
)
def _l0_kernel(xnA_hbm, xnB_hbm, send2d_hbm, recv2d_hbm, zeros_hbm,
               axA_hbm, axB_hbm, sidx, ridx, rows_v, acc, gsem, ssem):
    c = lax.axis_index("c")
    s = lax.axis_index("s")
    rpt = (_E // _C) // _NT
    nsb = rpt // _SBR

    @pl.when(c == 0)
    def _():
        _seg_body(xnA_hbm, axA_hbm, send2d_hbm, recv2d_hbm, zeros_hbm,
                  sidx, ridx, rows_v, acc, gsem, ssem, s, _K0, nsb,
                  _K0 // 2, _C)

    @pl.when(c == 1)
    def _():
        _seg_body(xnB_hbm, axB_hbm, send2d_hbm, recv2d_hbm, zeros_hbm,
                  sidx, ridx, rows_v, acc, gsem, ssem, s, _K0, nsb,
                  _K0 // 2, _C)


@functools.partial(
    pl.kernel,
    out_type=[jax.ShapeDtypeStruct((_NP, _D), _f32),
              jax.ShapeDtypeStruct((_NP, _D), _f32)],
    mesh=_mesh,
    scratch_types=[pltpu.VMEM((_SBR, _C1), jnp.int32),
                   pltpu.VMEM((_SBR, _C1), jnp.int32),
                   pltpu.VMEM((_K1 * _C1, _D), _f32),
                   pltpu.VMEM_SHARED((_NP, _D), _f32),
                   pltpu.SemaphoreType.DMA((_K1,)),
                   pltpu.SemaphoreType.DMA((_K1,))],
    compiler_params=pltpu.CompilerParams(use_tc_tiling_on_sc=False),
)
def _l1_kernel(hA_hbm, hB_hbm, send2d_hbm, recv2d_hbm, zeros_hbm,
               ahA_hbm, ahB_hbm, sidx, ridx, rows_v, acc, gsem, ssem):
    c = lax.axis_index("c")
    s = lax.axis_index("s")
    rpt = (_E // _C1) // _NT
    nsb = rpt // _SBR

    @pl.when(c == 0)
    def _():
        _seg_body(hA_hbm, ahA_hbm, send2d_hbm, recv2d_hbm, zeros_hbm,
                  sidx, ridx, rows_v, acc, gsem, ssem, s, _K1, nsb,
                  _K1 // 2, _C1)

    @pl.when(c == 1)
    def _():
        _seg_body(hB_hbm, ahB_hbm, send2d_hbm, recv2d_hbm, zeros_hbm,
                  sidx, ridx, rows_v, acc, gsem, ssem, s, _K1, nsb,
                  _K1 // 2, _C1)


_BR = 2048
_BRP = 2000


def _prep_body(nodes_ref, ds_ref, outa_ref, outb_ref):
    ns = lax.rsqrt(jnp.maximum(ds_ref[:, 0], 1.0))
    xn = nodes_ref[...] * ns[:, None]
    rows = xn.shape[0]
    pad7 = jnp.zeros((rows, _HALFW - 65), _f32)
    pad8 = jnp.zeros((rows, _HALFW - 64), _f32)
    outa_ref[...] = jnp.concatenate([xn[:, :64], ns[:, None], pad7], axis=1)
    outb_ref[...] = jnp.concatenate([xn[:, 64:], pad8], axis=1)


def _prep(nodes, ds16):
    return pl.pallas_call(
        _prep_body,
        grid=(_N // _BRP,),
        in_specs=[pl.BlockSpec((_BRP, _D), lambda i: (i, 0)),
                  pl.BlockSpec((_BRP, _DEGW), lambda i: (i, 0))],
        out_specs=[pl.BlockSpec((_BRP, _HALFW), lambda i: (i, 0)),
                   pl.BlockSpec((_BRP, _HALFW), lambda i: (i, 0))],
        out_shape=[jax.ShapeDtypeStruct((_N, _HALFW), _f32),
                   jax.ShapeDtypeStruct((_N, _HALFW), _f32)],
    )(nodes, ds16)


def _layer0_dense_body(axA_ref, axB_ref, dr_ref, ds_ref, w0a_ref, w0b_ref,
                       outa_ref, outb_ref):
    t = (jnp.dot(axA_ref[...], w0a_ref[...], preferred_element_type=_f32)
         + jnp.dot(axB_ref[...], w0b_ref[...], preferred_element_type=_f32))
    nr = lax.rsqrt(jnp.maximum(dr_ref[:, 0], 1.0))
    ns = lax.rsqrt(jnp.maximum(ds_ref[:, 0], 1.0))
    h0n = jnp.maximum(t * nr[:, None], 0.0) * ns[:, None]
    outa_ref[...] = h0n[:, :_D]
    outb_ref[...] = h0n[:, _D:]


def _layer0_dense(axA, axB, dr16, ds16, w0a, w0b):
    return pl.pallas_call(
        _layer0_dense_body,
        grid=(_NP // _BR,),
        in_specs=[pl.BlockSpec((_BR, _HALFW), lambda i: (i, 0)),
                  pl.BlockSpec((_BR, _HALFW), lambda i: (i, 0)),
                  pl.BlockSpec((_BR, _DEGW), lambda i: (i, 0)),
                  pl.BlockSpec((_BR, _DEGW), lambda i: (i, 0)),
                  pl.BlockSpec((_HALFW, _H), lambda i: (0, 0)),
                  pl.BlockSpec((_HALFW, _H), lambda i: (0, 0))],
        out_specs=[pl.BlockSpec((_BR, _D), lambda i: (i, 0)),
                   pl.BlockSpec((_BR, _D), lambda i: (i, 0))],
        out_shape=[jax.ShapeDtypeStruct((_NP, _D), _f32),
                   jax.ShapeDtypeStruct((_NP, _D), _f32)],
    )(axA, axB, dr16, ds16, w0a, w0b)


def _layer1_dense_body(ahA_ref, ahB_ref, axA_ref, axB_ref, dr_ref,
                       w1hi_ref, w1lo_ref, w1ba_ref, w1bb_ref,
                       w2_ref, b2_ref, inv_ref, out_ref, acc_ref):
    i = pl.program_id(0)
    g = (jnp.dot(ahA_ref[...], w1hi_ref[...], preferred_element_type=_f32)
         + jnp.dot(ahB_ref[...], w1lo_ref[...], preferred_element_type=_f32)
         + jnp.dot(axA_ref[...], w1ba_ref[...], preferred_element_type=_f32)
         + jnp.dot(axB_ref[...], w1bb_ref[...], preferred_element_type=_f32))
    nr = lax.rsqrt(jnp.maximum(dr_ref[:, 0], 1.0))
    h1 = jnp.maximum(g * nr[:, None], 0.0)
    psum = jnp.sum(h1, axis=0, keepdims=True)
    acc_ref[...] = jnp.where(i == 0, psum, acc_ref[...] + psum)

    @pl.when(i == pl.num_programs(0) - 1)
    def _():
        pooled = acc_ref[...] * inv_ref[0, 0]
        out_ref[...] = (jnp.dot(pooled, w2_ref[...], preferred_element_type=_f32)
                        + b2_ref[...])


def _layer1_dense(ahA, ahB, axA, axB, dr16, w1hi, w1lo, w1ba, w1bb,
                  w2, b2, inv):
    return pl.pallas_call(
        _layer1_dense_body,
        grid=(_NP // _BR,),
        in_specs=[pl.BlockSpec((_BR, _D), lambda i: (i, 0)),
                  pl.BlockSpec((_BR, _D), lambda i: (i, 0)),
                  pl.BlockSpec((_BR, _HALFW), lambda i: (i, 0)),
                  pl.BlockSpec((_BR, _HALFW), lambda i: (i, 0)),
                  pl.BlockSpec((_BR, _DEGW), lambda i: (i, 0)),
                  pl.BlockSpec((_D, _H), lambda i: (0, 0)),
                  pl.BlockSpec((_D, _H), lambda i: (0, 0)),
                  pl.BlockSpec((_HALFW, _H), lambda i: (0, 0)),
                  pl.BlockSpec((_HALFW, _H), lambda i: (0, 0)),
                  pl.BlockSpec((_H, _OUT), lambda i: (0, 0)),
                  pl.BlockSpec((1, _OUT), lambda i: (0, 0)),
                  pl.BlockSpec((1, 1), lambda i: (0, 0))],
        out_specs=pl.BlockSpec((1, _OUT), lambda i: (0, 0)),
        out_shape=jax.ShapeDtypeStruct((1, _OUT), _f32),
        scratch_shapes=[pltpu.VMEM((1, _H), _f32)],
    )(ahA, ahB, axA, axB, dr16, w1hi, w1lo, w1ba, w1bb, w2, b2, inv)


def kernel(nodes, senders, receivers, n_node, W0, b0, W1, b1, W2, b2):
    ones16 = jnp.ones((_C, _DEGW), _f32)
    zeros_deg = jnp.zeros((_NPT, _DEGW), _f32)
    zeros_half = jnp.zeros((_NPT, _HALFW), _f32)
    zeros_d = jnp.zeros((_NPT, _D), _f32)
    send2d = senders.reshape(_E // _C, _C)
    recv2d = receivers.reshape(_E // _C, _C)
    send2d1 = senders.reshape(_E // _C1, _C1)
    recv2d1 = receivers.reshape(_E // _C1, _C1)

    ds16, dr16 = _deg_kernel(send2d, recv2d, ones16, zeros_deg)
    xnA, xnB = _prep(nodes, ds16)
    axA, axB = _l0_kernel(xnA, xnB, send2d, recv2d, zeros_half)

    pad7 = jnp.zeros((_HALFW - 65, _H), _f32)
    pad8 = jnp.zeros((_HALFW - 64, _H), _f32)
    w0a = jnp.concatenate([W0[:64], b0[None, :], pad7], axis=0)
    w0b = jnp.concatenate([W0[64:], pad8], axis=0)
    hA, hB = _layer0_dense(axA, axB, dr16, ds16, w0a, w0b)

    ahA, ahB = _l1_kernel(hA, hB, send2d1, recv2d1, zeros_d)

    w1hi = W1[:_D]
    w1lo = W1[_D:_H]
    w1ba = jnp.concatenate([W1[_H:_H + 64], b1[None, :], pad7], axis=0)
    w1bb = jnp.concatenate([W1[_H + 64:], pad8], axis=0)
    inv = (1.0 / jnp.maximum(n_node.astype(_f32), 1.0)).reshape(1, 1)
    out = _layer1_dense(ahA, ahB, axA, axB, dr16, w1hi, w1lo, w1ba, w1bb,
                        W2, b2.reshape(1, _OUT), inv)
    return out.reshape(_OUT)


# speedup vs baseline: 1.0347x; 1.0347x over previous
import functools

import jax
import jax.numpy as jnp
from jax import lax
from jax.experimental import pallas as pl
from jax.experimental.pallas import tpu as pltpu
from jax.experimental.pallas import tpu_sc as plsc

_N = 10000
_E = 320000
_D = 128
_H = 256
_OUT = 128

_NT = 16
_NC = 2
_C = 80
_K0 = 10
_C1 = 80
_K1 = 4
_KD = 10
_SBR = 25
_HALFW = 72
_DEGW = 16
_NP = 10240
_NPT = _NP // _NT

_mesh = plsc.VectorSubcoreMesh(core_axis_name="c", subcore_axis_name="s")
_f32 = jnp.float32


@functools.partial(
    pl.kernel,
    out_type=[jax.ShapeDtypeStruct((_NP, _DEGW), _f32),
              jax.ShapeDtypeStruct((_NP, _DEGW), _f32)],
    mesh=_mesh,
    scratch_types=[pltpu.VMEM((_C, _DEGW), _f32),
                   pltpu.VMEM((_KD, _C), jnp.int32),
                   pltpu.VMEM_SHARED((_NP, _DEGW), _f32),
                   pltpu.SemaphoreType.DMA],
    compiler_params=pltpu.CompilerParams(use_tc_tiling_on_sc=False),
)
def _deg_kernel(send2d_hbm, recv2d_hbm, ones_hbm, zeros_hbm, ds_hbm, dr_hbm,
                ones_v, idx_v, acc, sem):
    c = lax.axis_index("c")
    s = lax.axis_index("s")
    pltpu.sync_copy(zeros_hbm, acc.at[pl.ds(s * _NPT, _NPT)])
    pltpu.sync_copy(ones_hbm, ones_v)
    plsc.subcore_barrier()

    rpt = (_E // _C) // _NT

    def hist(idx_hbm, out_hbm):
        def block(b, carry):
            r0 = s * rpt + b * _KD
            pltpu.sync_copy(idx_hbm.at[pl.ds(r0, _KD)], idx_v)
            adds = [pltpu.async_copy(ones_v, acc.at[idx_v.at[j]], sem, add=True)
                    for j in range(_KD)]
            for t in adds:
                t.wait()
            return carry
        lax.fori_loop(0, rpt // _KD, block, 0)
        plsc.subcore_barrier()
        pltpu.sync_copy(acc.at[pl.ds(s * _NPT, _NPT)],
                        out_hbm.at[pl.ds(s * _NPT, _NPT)])

    @pl.when(c == 0)
    def _():
        hist(send2d_hbm, ds_hbm)

    @pl.when(c == 1)
    def _():
        hist(recv2d_hbm, dr_hbm)


def _seg_body(table_hbm, out_hbm, send2d, recv2d, zeros_hbm,
              sidx, ridx, rows_v, acc, gsems, ssems, s, k, nsb, lag, csz):
    pltpu.sync_copy(zeros_hbm, acc.at[pl.ds(s * _NPT, _NPT)])
    plsc.subcore_barrier()
    row_base = s * (nsb * _SBR)

    def superblock(sb, carry):
        r0 = row_base + sb * _SBR
        pltpu.sync_copy(send2d.at[pl.ds(r0, _SBR)], sidx)
        pltpu.sync_copy(recv2d.at[pl.ds(r0, _SBR)], ridx)

        gdesc = [None] * _SBR
        sdesc = [None] * _SBR

        def fire_scatter(t):
            slot = t % k
            gdesc[t].wait()
            sdesc[t] = pltpu.async_copy(rows_v.at[pl.ds(slot * csz, csz)],
                                        acc.at[ridx.at[t]], ssems.at[slot],
                                        add=True)

        for j in range(_SBR):
            slot = j % k
            if j >= k:
                sdesc[j - k].wait()
            gdesc[j] = pltpu.async_copy(table_hbm.at[sidx.at[j]],
                                        rows_v.at[pl.ds(slot * csz, csz)],
                                        gsems.at[slot])
            if j >= lag:
                fire_scatter(j - lag)
        for t in range(_SBR - lag, _SBR):
            fire_scatter(t)
        for t in range(_SBR - k, _SBR):
            sdesc[t].wait()
        return carry

    lax.fori_loop(0, nsb, superblock, 0)
    plsc.subcore_barrier()
    pltpu.sync_copy(acc.at[pl.ds(s * _NPT, _NPT)],
                    out_hbm.at[pl.ds(s * _NPT, _NPT)])


@functools.partial(
    pl.kernel,
    out_type=[jax.ShapeDtypeStruct((_NP, _HALFW), _f32),
              jax.ShapeDtypeStruct((_NP, _HALFW), _f32)],
    mesh=_mesh,
    scratch_types=[pltpu.VMEM((_SBR, _C), jnp.int32),
                   pltpu.VMEM((_SBR, _C), jnp.int32),
                   pltpu.VMEM((_K0 * _C, _HALFW), _f32),
                   pltpu.VMEM_SHARED((_NP, _HALFW), _f32),
                   pltpu.SemaphoreType.DMA((_K0,)),
                   pltpu.SemaphoreType.DMA((_K0,))],
    compiler_params=pltpu.CompilerParams(use_tc_tiling_on_sc=False),
)
def _l0_kernel(xnA_hbm, xnB_hbm, send2d_hbm, recv2d_hbm, zeros_hbm,
               axA_hbm, axB_hbm, sidx, ridx, rows_v, acc, gsem, ssem):
    c = lax.axis_index("c")
    s = lax.axis_index("s")
    rpt = (_E // _C) // _NT
    nsb = rpt // _SBR

    @pl.when(c == 0)
    def _():
        _seg_body(xnA_hbm, axA_hbm, send2d_hbm, recv2d_hbm, zeros_hbm,
                  sidx, ridx, rows_v, acc, gsem, ssem, s, _K0, nsb,
                  _K0 // 2, _C)

    @pl.when(c == 1)
    def _():
        _seg_body(xnB_hbm, axB_hbm, send2d_hbm, recv2d_hbm, zeros_hbm,
                  sidx, ridx, rows_v, acc, gsem, ssem, s, _K0, nsb,
                  _K0 // 2, _C)


@functools.partial(
    pl.kernel,
    out_type=[jax.ShapeDtypeStruct((_NP, _D), _f32),
              jax.ShapeDtypeStruct((_NP, _D), _f32)],
    mesh=_mesh,
    scratch_types=[pltpu.VMEM((_SBR, _C1), jnp.int32),
                   pltpu.VMEM((_SBR, _C1), jnp.int32),
                   pltpu.VMEM((_K1 * _C1, _D), _f32),
                   pltpu.VMEM_SHARED((_NP, _D), _f32),
                   pltpu.SemaphoreType.DMA((_K1,)),
                   pltpu.SemaphoreType.DMA((_K1,))],
    compiler_params=pltpu.CompilerParams(use_tc_tiling_on_sc=False),
)
def _l1_kernel(hA_hbm, hB_hbm, send2d_hbm, recv2d_hbm, zeros_hbm,
               ahA_hbm, ahB_hbm, sidx, ridx, rows_v, acc, gsem, ssem):
    c = lax.axis_index("c")
    s = lax.axis_index("s")
    rpt = (_E // _C1) // _NT
    nsb = rpt // _SBR

    @pl.when(c == 0)
    def _():
        _seg_body(hA_hbm, ahA_hbm, send2d_hbm, recv2d_hbm, zeros_hbm,
                  sidx, ridx, rows_v, acc, gsem, ssem, s, _K1, nsb,
                  _K1 // 2, _C1)

    @pl.when(c == 1)
    def _():
        _seg_body(hB_hbm, ahB_hbm, send2d_hbm, recv2d_hbm, zeros_hbm,
                  sidx, ridx, rows_v, acc, gsem, ssem, s, _K1, nsb,
                  _K1 // 2, _C1)


_BR = 2048
_BRP = 2000


def _prep_body(nodes_ref, ds_ref, outa_ref, outb_ref):
    ns = lax.rsqrt(jnp.maximum(ds_ref[:, 0], 1.0))
    xn = nodes_ref[...] * ns[:, None]
    rows = xn.shape[0]
    pad7 = jnp.zeros((rows, _HALFW - 65), _f32)
    pad8 = jnp.zeros((rows, _HALFW - 64), _f32)
    outa_ref[...] = jnp.concatenate([xn[:, :64], ns[:, None], pad7], axis=1)
    outb_ref[...] = jnp.concatenate([xn[:, 64:], pad8], axis=1)


def _prep(nodes, ds16):
    return pl.pallas_call(
        _prep_body,
        grid=(_N // _BRP,),
        in_specs=[pl.BlockSpec((_BRP, _D), lambda i: (i, 0)),
                  pl.BlockSpec((_BRP, _DEGW), lambda i: (i, 0))],
        out_specs=[pl.BlockSpec((_BRP, _HALFW), lambda i: (i, 0)),
                   pl.BlockSpec((_BRP, _HALFW), lambda i: (i, 0))],
        out_shape=[jax.ShapeDtypeStruct((_N, _HALFW), _f32),
                   jax.ShapeDtypeStruct((_N, _HALFW), _f32)],
    )(nodes, ds16)


def _layer0_dense_body(axA_ref, axB_ref, dr_ref, ds_ref, w0a_ref, w0b_ref,
                       outa_ref, outb_ref):
    t = (jnp.dot(axA_ref[...], w0a_ref[...], preferred_element_type=_f32)
         + jnp.dot(axB_ref[...], w0b_ref[...], preferred_element_type=_f32))
    nr = lax.rsqrt(jnp.maximum(dr_ref[:, 0], 1.0))
    ns = lax.rsqrt(jnp.maximum(ds_ref[:, 0], 1.0))
    h0n = jnp.maximum(t * nr[:, None], 0.0) * ns[:, None]
    outa_ref[...] = h0n[:, :_D]
    outb_ref[...] = h0n[:, _D:]


def _layer0_dense(axA, axB, dr16, ds16, w0a, w0b):
    return pl.pallas_call(
        _layer0_dense_body,
        grid=(_NP // _BR,),
        in_specs=[pl.BlockSpec((_BR, _HALFW), lambda i: (i, 0)),
                  pl.BlockSpec((_BR, _HALFW), lambda i: (i, 0)),
                  pl.BlockSpec((_BR, _DEGW), lambda i: (i, 0)),
                  pl.BlockSpec((_BR, _DEGW), lambda i: (i, 0)),
                  pl.BlockSpec((_HALFW, _H), lambda i: (0, 0)),
                  pl.BlockSpec((_HALFW, _H), lambda i: (0, 0))],
        out_specs=[pl.BlockSpec((_BR, _D), lambda i: (i, 0)),
                   pl.BlockSpec((_BR, _D), lambda i: (i, 0))],
        out_shape=[jax.ShapeDtypeStruct((_NP, _D), _f32),
                   jax.ShapeDtypeStruct((_NP, _D), _f32)],
    )(axA, axB, dr16, ds16, w0a, w0b)


def _layer1_dense_body(ahA_ref, ahB_ref, axA_ref, axB_ref, dr_ref,
                       w1hi_ref, w1lo_ref, w1ba_ref, w1bb_ref,
                       w2_ref, b2_ref, inv_ref, out_ref, acc_ref):
    i = pl.program_id(0)
    g = (jnp.dot(ahA_ref[...], w1hi_ref[...], preferred_element_type=_f32)
         + jnp.dot(ahB_ref[...], w1lo_ref[...], preferred_element_type=_f32)
         + jnp.dot(axA_ref[...], w1ba_ref[...], preferred_element_type=_f32)
         + jnp.dot(axB_ref[...], w1bb_ref[...], preferred_element_type=_f32))
    nr = lax.rsqrt(jnp.maximum(dr_ref[:, 0], 1.0))
    h1 = jnp.maximum(g * nr[:, None], 0.0)
    psum = jnp.sum(h1, axis=0, keepdims=True)
    acc_ref[...] = jnp.where(i == 0, psum, acc_ref[...] + psum)

    @pl.when(i == pl.num_programs(0) - 1)
    def _():
        pooled = acc_ref[...] * inv_ref[0, 0]
        out_ref[...] = (jnp.dot(pooled, w2_ref[...], preferred_element_type=_f32)
                        + b2_ref[...])


def _layer1_dense(ahA, ahB, axA, axB, dr16, w1hi, w1lo, w1ba, w1bb,
                  w2, b2, inv):
    return pl.pallas_call(
        _layer1_dense_body,
        grid=(_NP // _BR,),
        in_specs=[pl.BlockSpec((_BR, _D), lambda i: (i, 0)),
                  pl.BlockSpec((_BR, _D), lambda i: (i, 0)),
                  pl.BlockSpec((_BR, _HALFW), lambda i: (i, 0)),
                  pl.BlockSpec((_BR, _HALFW), lambda i: (i, 0)),
                  pl.BlockSpec((_BR, _DEGW), lambda i: (i, 0)),
                  pl.BlockSpec((_D, _H), lambda i: (0, 0)),
                  pl.BlockSpec((_D, _H), lambda i: (0, 0)),
                  pl.BlockSpec((_HALFW, _H), lambda i: (0, 0)),
                  pl.BlockSpec((_HALFW, _H), lambda i: (0, 0)),
                  pl.BlockSpec((_H, _OUT), lambda i: (0, 0)),
                  pl.BlockSpec((1, _OUT), lambda i: (0, 0)),
                  pl.BlockSpec((1, 1), lambda i: (0, 0))],
        out_specs=pl.BlockSpec((1, _OUT), lambda i: (0, 0)),
        out_shape=jax.ShapeDtypeStruct((1, _OUT), _f32),
        scratch_shapes=[pltpu.VMEM((1, _H), _f32)],
    )(ahA, ahB, axA, axB, dr16, w1hi, w1lo, w1ba, w1bb, w2, b2, inv)


def kernel(nodes, senders, receivers, n_node, W0, b0, W1, b1, W2, b2):
    ones16 = jnp.ones((_C, _DEGW), _f32)
    zeros_deg = jnp.zeros((_NPT, _DEGW), _f32)
    zeros_half = jnp.zeros((_NPT, _HALFW), _f32)
    zeros_d = jnp.zeros((_NPT, _D), _f32)
    send2d = senders.reshape(_E // _C, _C)
    recv2d = receivers.reshape(_E // _C, _C)
    send2d1 = senders.reshape(_E // _C1, _C1)
    recv2d1 = receivers.reshape(_E // _C1, _C1)

    ds16, dr16 = _deg_kernel(send2d, recv2d, ones16, zeros_deg)
    xnA, xnB = _prep(nodes, ds16)
    axA, axB = _l0_kernel(xnA, xnB, send2d, recv2d, zeros_half)

    pad7 = jnp.zeros((_HALFW - 65, _H), _f32)
    pad8 = jnp.zeros((_HALFW - 64, _H), _f32)
    w0a = jnp.concatenate([W0[:64], b0[None, :], pad7], axis=0)
    w0b = jnp.concatenate([W0[64:], pad8], axis=0)
    hA, hB = _layer0_dense(axA, axB, dr16, ds16, w0a, w0b)

    ahA, ahB = _l1_kernel(hA, hB, send2d1, recv2d1, zeros_d)

    w1hi = W1[:_D]
    w1lo = W1[_D:_H]
    w1ba = jnp.concatenate([W1[_H:_H + 64], b1[None, :], pad7], axis=0)
    w1bb = jnp.concatenate([W1[_H + 64:], pad8], axis=0)
    inv = (1.0 / jnp.maximum(n_node.astype(_f32), 1.0)).reshape(1, 1)
    out = _layer1_dense(ahA, ahB, axA, axB, dr16, w1hi, w1lo, w1ba, w1bb,
                        W2, b2.reshape(1, _OUT), inv)
    return out.reshape(_OUT)


# speedup vs baseline: 1.0685x; 1.0327x over previous
import functools

import jax
import jax.numpy as jnp
from jax import lax
from jax.experimental import pallas as pl
from jax.experimental.pallas import tpu as pltpu
from jax.experimental.pallas import tpu_sc as plsc

_N = 10000
_E = 320000
_D = 128
_H = 256
_OUT = 128

_NT = 16
_NC = 2
_C = 80
_K0 = 10
_C1 = 80
_K1 = 4
_SBR = 25
_HALFW = 72
_DEGW = 8
_NP = 10240
_NPT = _NP // _NT

_mesh = plsc.VectorSubcoreMesh(core_axis_name="c", subcore_axis_name="s")
_f32 = jnp.float32


@functools.partial(
    pl.kernel,
    out_type=[jax.ShapeDtypeStruct((_NP, _DEGW), _f32),
              jax.ShapeDtypeStruct((_NP, _DEGW), _f32)],
    mesh=_mesh,
    scratch_types=[pltpu.VMEM((_C, _DEGW), _f32),
                   pltpu.VMEM((_SBR, _C), jnp.int32),
                   pltpu.VMEM_SHARED((_NP, _DEGW), _f32),
                   pltpu.SemaphoreType.DMA],
    compiler_params=pltpu.CompilerParams(use_tc_tiling_on_sc=False),
)
def _deg_kernel(send2d_hbm, recv2d_hbm, ones_hbm, zeros_hbm, ds_hbm, dr_hbm,
                ones_v, idx_v, acc, sem):
    c = lax.axis_index("c")
    s = lax.axis_index("s")
    pltpu.sync_copy(zeros_hbm, acc.at[pl.ds(s * _NPT, _NPT)])
    pltpu.sync_copy(ones_hbm, ones_v)
    plsc.subcore_barrier()

    rpt = (_E // _C) // _NT
    nsb = rpt // _SBR

    def hist(idx_hbm, out_hbm):
        def superblock(sb, carry):
            r0 = s * rpt + sb * _SBR
            pltpu.sync_copy(idx_hbm.at[pl.ds(r0, _SBR)], idx_v)
            adds = [pltpu.async_copy(ones_v, acc.at[idx_v.at[j]], sem,
                                     add=True)
                    for j in range(_SBR)]
            for t in adds:
                t.wait()
            return carry
        lax.fori_loop(0, nsb, superblock, 0)
        plsc.subcore_barrier()
        pltpu.sync_copy(acc.at[pl.ds(s * _NPT, _NPT)],
                        out_hbm.at[pl.ds(s * _NPT, _NPT)])

    @pl.when(c == 0)
    def _():
        hist(send2d_hbm, ds_hbm)

    @pl.when(c == 1)
    def _():
        hist(recv2d_hbm, dr_hbm)


def _seg_body(table_hbm, out_hbm, send2d, recv2d, zeros_hbm,
              sidx, ridx, rows_v, acc, gsems, ssems, s, k, nsb, lag, csz):
    pltpu.sync_copy(zeros_hbm, acc.at[pl.ds(s * _NPT, _NPT)])
    plsc.subcore_barrier()
    row_base = s * (nsb * _SBR)

    def superblock(sb, carry):
        r0 = row_base + sb * _SBR
        pltpu.sync_copy(send2d.at[pl.ds(r0, _SBR)], sidx)
        pltpu.sync_copy(recv2d.at[pl.ds(r0, _SBR)], ridx)

        gdesc = [None] * _SBR
        sdesc = [None] * _SBR

        def fire_scatter(t):
            slot = t % k
            gdesc[t].wait()
            sdesc[t] = pltpu.async_copy(rows_v.at[pl.ds(slot * csz, csz)],
                                        acc.at[ridx.at[t]], ssems.at[slot],
                                        add=True)

        for j in range(_SBR):
            slot = j % k
            if j >= k:
                sdesc[j - k].wait()
            gdesc[j] = pltpu.async_copy(table_hbm.at[sidx.at[j]],
                                        rows_v.at[pl.ds(slot * csz, csz)],
                                        gsems.at[slot])
            if j >= lag:
                fire_scatter(j - lag)
        for t in range(_SBR - lag, _SBR):
            fire_scatter(t)
        for t in range(_SBR - k, _SBR):
            sdesc[t].wait()
        return carry

    lax.fori_loop(0, nsb, superblock, 0)
    plsc.subcore_barrier()
    pltpu.sync_copy(acc.at[pl.ds(s * _NPT, _NPT)],
                    out_hbm.at[pl.ds(s * _NPT, _NPT)])


@functools.partial(
    pl.kernel,
    out_type=[jax.ShapeDtypeStruct((_NP, _HALFW), _f32),
              jax.ShapeDtypeStruct((_NP, _HALFW), _f32)],
    mesh=_mesh,
    scratch_types=[pltpu.VMEM((_SBR, _C), jnp.int32),
                   pltpu.VMEM((_SBR, _C), jnp.int32),
                   pltpu.VMEM((_K0 * _C, _HALFW), _f32),
                   pltpu.VMEM_SHARED((_NP, _HALFW), _f32),
                   pltpu.SemaphoreType.DMA((_K0,)),
                   pltpu.SemaphoreType.DMA((_K0,))],
    compiler_params=pltpu.CompilerParams(use_tc_tiling_on_sc=False),
)
def _l0_kernel(xnA_hbm, xnB_hbm, send2d_hbm, recv2d_hbm, zeros_hbm,
               axA_hbm, axB_hbm, sidx, ridx, rows_v, acc, gsem, ssem):
    c = lax.axis_index("c")
    s = lax.axis_index("s")
    rpt = (_E // _C) // _NT
    nsb = rpt // _SBR

    @pl.when(c == 0)
    def _():
        _seg_body(xnA_hbm, axA_hbm, send2d_hbm, recv2d_hbm, zeros_hbm,
                  sidx, ridx, rows_v, acc, gsem, ssem, s, _K0, nsb,
                  _K0 // 2, _C)

    @pl.when(c == 1)
    def _():
        _seg_body(xnB_hbm, axB_hbm, send2d_hbm, recv2d_hbm, zeros_hbm,
                  sidx, ridx, rows_v, acc, gsem, ssem, s, _K0, nsb,
                  _K0 // 2, _C)


@functools.partial(
    pl.kernel,
    out_type=[jax.ShapeDtypeStruct((_NP, _D), _f32),
              jax.ShapeDtypeStruct((_NP, _D), _f32)],
    mesh=_mesh,
    scratch_types=[pltpu.VMEM((_SBR, _C1), jnp.int32),
                   pltpu.VMEM((_SBR, _C1), jnp.int32),
                   pltpu.VMEM((_K1 * _C1, _D), _f32),
                   pltpu.VMEM_SHARED((_NP, _D), _f32),
                   pltpu.SemaphoreType.DMA((_K1,)),
                   pltpu.SemaphoreType.DMA((_K1,))],
    compiler_params=pltpu.CompilerParams(use_tc_tiling_on_sc=False),
)
def _l1_kernel(hA_hbm, hB_hbm, send2d_hbm, recv2d_hbm, zeros_hbm,
               ahA_hbm, ahB_hbm, sidx, ridx, rows_v, acc, gsem, ssem):
    c = lax.axis_index("c")
    s = lax.axis_index("s")
    rpt = (_E // _C1) // _NT
    nsb = rpt // _SBR

    @pl.when(c == 0)
    def _():
        _seg_body(hA_hbm, ahA_hbm, send2d_hbm, recv2d_hbm, zeros_hbm,
                  sidx, ridx, rows_v, acc, gsem, ssem, s, _K1, nsb,
                  _K1 // 2, _C1)

    @pl.when(c == 1)
    def _():
        _seg_body(hB_hbm, ahB_hbm, send2d_hbm, recv2d_hbm, zeros_hbm,
                  sidx, ridx, rows_v, acc, gsem, ssem, s, _K1, nsb,
                  _K1 // 2, _C1)


_BR = 2048
_BRP = 2000


def _prep_body(nodes_ref, ds_ref, outa_ref, outb_ref):
    ns = lax.rsqrt(jnp.maximum(ds_ref[:, 0], 1.0))
    xn = nodes_ref[...] * ns[:, None]
    rows = xn.shape[0]
    pad7 = jnp.zeros((rows, _HALFW - 65), _f32)
    pad8 = jnp.zeros((rows, _HALFW - 64), _f32)
    outa_ref[...] = jnp.concatenate([xn[:, :64], ns[:, None], pad7], axis=1)
    outb_ref[...] = jnp.concatenate([xn[:, 64:], pad8], axis=1)


def _prep(nodes, ds16):
    return pl.pallas_call(
        _prep_body,
        grid=(_N // _BRP,),
        in_specs=[pl.BlockSpec((_BRP, _D), lambda i: (i, 0)),
                  pl.BlockSpec((_BRP, _DEGW), lambda i: (i, 0))],
        out_specs=[pl.BlockSpec((_BRP, _HALFW), lambda i: (i, 0)),
                   pl.BlockSpec((_BRP, _HALFW), lambda i: (i, 0))],
        out_shape=[jax.ShapeDtypeStruct((_N, _HALFW), _f32),
                   jax.ShapeDtypeStruct((_N, _HALFW), _f32)],
    )(nodes, ds16)


def _layer0_dense_body(axA_ref, axB_ref, dr_ref, ds_ref, w0a_ref, w0b_ref,
                       outa_ref, outb_ref):
    t = (jnp.dot(axA_ref[...], w0a_ref[...], preferred_element_type=_f32)
         + jnp.dot(axB_ref[...], w0b_ref[...], preferred_element_type=_f32))
    nr = lax.rsqrt(jnp.maximum(dr_ref[:, 0], 1.0))
    ns = lax.rsqrt(jnp.maximum(ds_ref[:, 0], 1.0))
    h0n = jnp.maximum(t * nr[:, None], 0.0) * ns[:, None]
    outa_ref[...] = h0n[:, :_D]
    outb_ref[...] = h0n[:, _D:]


def _layer0_dense(axA, axB, dr16, ds16, w0a, w0b):
    return pl.pallas_call(
        _layer0_dense_body,
        grid=(_NP // _BR,),
        in_specs=[pl.BlockSpec((_BR, _HALFW), lambda i: (i, 0)),
                  pl.BlockSpec((_BR, _HALFW), lambda i: (i, 0)),
                  pl.BlockSpec((_BR, _DEGW), lambda i: (i, 0)),
                  pl.BlockSpec((_BR, _DEGW), lambda i: (i, 0)),
                  pl.BlockSpec((_HALFW, _H), lambda i: (0, 0)),
                  pl.BlockSpec((_HALFW, _H), lambda i: (0, 0))],
        out_specs=[pl.BlockSpec((_BR, _D), lambda i: (i, 0)),
                   pl.BlockSpec((_BR, _D), lambda i: (i, 0))],
        out_shape=[jax.ShapeDtypeStruct((_NP, _D), _f32),
                   jax.ShapeDtypeStruct((_NP, _D), _f32)],
    )(axA, axB, dr16, ds16, w0a, w0b)


def _layer1_dense_body(ahA_ref, ahB_ref, axA_ref, axB_ref, dr_ref,
                       w1hi_ref, w1lo_ref, w1ba_ref, w1bb_ref,
                       w2_ref, b2_ref, inv_ref, out_ref, acc_ref):
    i = pl.program_id(0)
    g = (jnp.dot(ahA_ref[...], w1hi_ref[...], preferred_element_type=_f32)
         + jnp.dot(ahB_ref[...], w1lo_ref[...], preferred_element_type=_f32)
         + jnp.dot(axA_ref[...], w1ba_ref[...], preferred_element_type=_f32)
         + jnp.dot(axB_ref[...], w1bb_ref[...], preferred_element_type=_f32))
    nr = lax.rsqrt(jnp.maximum(dr_ref[:, 0], 1.0))
    h1 = jnp.maximum(g * nr[:, None], 0.0)
    psum = jnp.sum(h1, axis=0, keepdims=True)
    acc_ref[...] = jnp.where(i == 0, psum, acc_ref[...] + psum)

    @pl.when(i == pl.num_programs(0) - 1)
    def _():
        pooled = acc_ref[...] * inv_ref[0, 0]
        out_ref[...] = (jnp.dot(pooled, w2_ref[...], preferred_element_type=_f32)
                        + b2_ref[...])


def _layer1_dense(ahA, ahB, axA, axB, dr16, w1hi, w1lo, w1ba, w1bb,
                  w2, b2, inv):
    return pl.pallas_call(
        _layer1_dense_body,
        grid=(_NP // _BR,),
        in_specs=[pl.BlockSpec((_BR, _D), lambda i: (i, 0)),
                  pl.BlockSpec((_BR, _D), lambda i: (i, 0)),
                  pl.BlockSpec((_BR, _HALFW), lambda i: (i, 0)),
                  pl.BlockSpec((_BR, _HALFW), lambda i: (i, 0)),
                  pl.BlockSpec((_BR, _DEGW), lambda i: (i, 0)),
                  pl.BlockSpec((_D, _H), lambda i: (0, 0)),
                  pl.BlockSpec((_D, _H), lambda i: (0, 0)),
                  pl.BlockSpec((_HALFW, _H), lambda i: (0, 0)),
                  pl.BlockSpec((_HALFW, _H), lambda i: (0, 0)),
                  pl.BlockSpec((_H, _OUT), lambda i: (0, 0)),
                  pl.BlockSpec((1, _OUT), lambda i: (0, 0)),
                  pl.BlockSpec((1, 1), lambda i: (0, 0))],
        out_specs=pl.BlockSpec((1, _OUT), lambda i: (0, 0)),
        out_shape=jax.ShapeDtypeStruct((1, _OUT), _f32),
        scratch_shapes=[pltpu.VMEM((1, _H), _f32)],
    )(ahA, ahB, axA, axB, dr16, w1hi, w1lo, w1ba, w1bb, w2, b2, inv)


def kernel(nodes, senders, receivers, n_node, W0, b0, W1, b1, W2, b2):
    ones16 = jnp.ones((_C, _DEGW), _f32)
    zeros_deg = jnp.zeros((_NPT, _DEGW), _f32)
    zeros_half = jnp.zeros((_NPT, _HALFW), _f32)
    zeros_d = jnp.zeros((_NPT, _D), _f32)
    send2d = senders.reshape(_E // _C, _C)
    recv2d = receivers.reshape(_E // _C, _C)
    send2d1 = senders.reshape(_E // _C1, _C1)
    recv2d1 = receivers.reshape(_E // _C1, _C1)

    ds16, dr16 = _deg_kernel(send2d, recv2d, ones16, zeros_deg)
    xnA, xnB = _prep(nodes, ds16)
    axA, axB = _l0_kernel(xnA, xnB, send2d, recv2d, zeros_half)

    pad7 = jnp.zeros((_HALFW - 65, _H), _f32)
    pad8 = jnp.zeros((_HALFW - 64, _H), _f32)
    w0a = jnp.concatenate([W0[:64], b0[None, :], pad7], axis=0)
    w0b = jnp.concatenate([W0[64:], pad8], axis=0)
    hA, hB = _layer0_dense(axA, axB, dr16, ds16, w0a, w0b)

    ahA, ahB = _l1_kernel(hA, hB, send2d1, recv2d1, zeros_d)

    w1hi = W1[:_D]
    w1lo = W1[_D:_H]
    w1ba = jnp.concatenate([W1[_H:_H + 64], b1[None, :], pad7], axis=0)
    w1bb = jnp.concatenate([W1[_H + 64:], pad8], axis=0)
    inv = (1.0 / jnp.maximum(n_node.astype(_f32), 1.0)).reshape(1, 1)
    out = _layer1_dense(ahA, ahB, axA, axB, dr16, w1hi, w1lo, w1ba, w1bb,
                        W2, b2.reshape(1, _OUT), inv)
    return out.reshape(_OUT)


# speedup vs baseline: 1.0972x; 1.0268x over previous
import functools

import jax
import jax.numpy as jnp
from jax import lax
from jax.experimental import pallas as pl
from jax.experimental.pallas import tpu as pltpu
from jax.experimental.pallas import tpu_sc as plsc

_N = 10000
_E = 320000
_D = 128
_H = 256
_OUT = 128

_NT = 16
_NC = 2
_C = 80
_K0 = 10
_C1 = 80
_K1 = 4
_SBR = 25
_HALFW = 72
_DEGW = 8
_NP = 10240
_NPT = _NP // _NT

_mesh = plsc.VectorSubcoreMesh(core_axis_name="c", subcore_axis_name="s")
_f32 = jnp.float32


@functools.partial(
    pl.kernel,
    out_type=[jax.ShapeDtypeStruct((_NP, _DEGW), _f32),
              jax.ShapeDtypeStruct((_NP, _DEGW), _f32)],
    mesh=_mesh,
    scratch_types=[pltpu.VMEM((_C, _DEGW), _f32),
                   pltpu.VMEM((_SBR, _C), jnp.int32),
                   pltpu.VMEM_SHARED((_NP, _DEGW), _f32),
                   pltpu.SemaphoreType.DMA],
    compiler_params=pltpu.CompilerParams(use_tc_tiling_on_sc=False),
)
def _deg_kernel(send2d_hbm, recv2d_hbm, ones_hbm, zeros_hbm, ds_hbm, dr_hbm,
                ones_v, idx_v, acc, sem):
    c = lax.axis_index("c")
    s = lax.axis_index("s")
    pltpu.sync_copy(zeros_hbm, acc.at[pl.ds(s * _NPT, _NPT)])
    pltpu.sync_copy(ones_hbm, ones_v)
    plsc.subcore_barrier()

    rpt = (_E // _C) // _NT
    nsb = rpt // _SBR

    def hist(idx_hbm, out_hbm):
        def superblock(sb, carry):
            r0 = s * rpt + sb * _SBR
            pltpu.sync_copy(idx_hbm.at[pl.ds(r0, _SBR)], idx_v)
            adds = [pltpu.async_copy(ones_v, acc.at[idx_v.at[j]], sem,
                                     add=True)
                    for j in range(_SBR)]
            for t in adds:
                t.wait()
            return carry
        lax.fori_loop(0, nsb, superblock, 0)
        plsc.subcore_barrier()
        pltpu.sync_copy(acc.at[pl.ds(s * _NPT, _NPT)],
                        out_hbm.at[pl.ds(s * _NPT, _NPT)])

    @pl.when(c == 0)
    def _():
        hist(send2d_hbm, ds_hbm)

    @pl.when(c == 1)
    def _():
        hist(recv2d_hbm, dr_hbm)


def _seg_body(table_hbm, out_hbm, send2d, recv2d, zeros_hbm,
              sidx, ridx, rows_v, acc, gsems, ssems, s, k, nsb, lag, csz):
    pltpu.sync_copy(zeros_hbm, acc.at[pl.ds(s * _NPT, _NPT)])
    plsc.subcore_barrier()
    row_base = s * (nsb * _SBR)

    def superblock(sb, carry):
        r0 = row_base + sb * _SBR
        pltpu.sync_copy(send2d.at[pl.ds(r0, _SBR)], sidx)
        pltpu.sync_copy(recv2d.at[pl.ds(r0, _SBR)], ridx)

        gdesc = [None] * _SBR
        sdesc = [None] * _SBR

        def fire_scatter(t):
            slot = t % k
            gdesc[t].wait()
            sdesc[t] = pltpu.async_copy(rows_v.at[pl.ds(slot * csz, csz)],
                                        acc.at[ridx.at[t]], ssems.at[slot],
                                        add=True)

        for j in range(_SBR):
            slot = j % k
            if j >= k:
                sdesc[j - k].wait()
            gdesc[j] = pltpu.async_copy(table_hbm.at[sidx.at[j]],
                                        rows_v.at[pl.ds(slot * csz, csz)],
                                        gsems.at[slot])
            if j >= lag:
                fire_scatter(j - lag)
        for t in range(_SBR - lag, _SBR):
            fire_scatter(t)
        for t in range(_SBR - k, _SBR):
            sdesc[t].wait()
        return carry

    lax.fori_loop(0, nsb, superblock, 0)
    plsc.subcore_barrier()
    pltpu.sync_copy(acc.at[pl.ds(s * _NPT, _NPT)],
                    out_hbm.at[pl.ds(s * _NPT, _NPT)])


@functools.partial(
    pl.kernel,
    out_type=[jax.ShapeDtypeStruct((_NP, _HALFW), _f32),
              jax.ShapeDtypeStruct((_NP, _HALFW), _f32)],
    mesh=_mesh,
    scratch_types=[pltpu.VMEM((_SBR, _C), jnp.int32),
                   pltpu.VMEM((_SBR, _C), jnp.int32),
                   pltpu.VMEM((_K0 * _C, _HALFW), _f32),
                   pltpu.VMEM_SHARED((_NP, _HALFW), _f32),
                   pltpu.SemaphoreType.DMA((_K0,)),
                   pltpu.SemaphoreType.DMA((_K0,))],
    compiler_params=pltpu.CompilerParams(use_tc_tiling_on_sc=False),
)
def _l0_kernel(xnA_hbm, xnB_hbm, send2d_hbm, recv2d_hbm, zeros_hbm,
               axA_hbm, axB_hbm, sidx, ridx, rows_v, acc, gsem, ssem):
    c = lax.axis_index("c")
    s = lax.axis_index("s")
    rpt = (_E // _C) // _NT
    nsb = rpt // _SBR

    @pl.when(c == 0)
    def _():
        _seg_body(xnA_hbm, axA_hbm, send2d_hbm, recv2d_hbm, zeros_hbm,
                  sidx, ridx, rows_v, acc, gsem, ssem, s, _K0, nsb,
                  _K0 // 2, _C)

    @pl.when(c == 1)
    def _():
        _seg_body(xnB_hbm, axB_hbm, send2d_hbm, recv2d_hbm, zeros_hbm,
                  sidx, ridx, rows_v, acc, gsem, ssem, s, _K0, nsb,
                  _K0 // 2, _C)


@functools.partial(
    pl.kernel,
    out_type=[jax.ShapeDtypeStruct((_NP, _D), _f32),
              jax.ShapeDtypeStruct((_NP, _D), _f32)],
    mesh=_mesh,
    scratch_types=[pltpu.VMEM((_SBR, _C1), jnp.int32),
                   pltpu.VMEM((_SBR, _C1), jnp.int32),
                   pltpu.VMEM((_K1 * _C1, _D), _f32),
                   pltpu.VMEM_SHARED((_NP, _D), _f32),
                   pltpu.SemaphoreType.DMA((_K1,)),
                   pltpu.SemaphoreType.DMA((_K1,))],
    compiler_params=pltpu.CompilerParams(use_tc_tiling_on_sc=False),
)
def _l1_kernel(hA_hbm, hB_hbm, send2d_hbm, recv2d_hbm, zeros_hbm,
               ahA_hbm, ahB_hbm, sidx, ridx, rows_v, acc, gsem, ssem):
    c = lax.axis_index("c")
    s = lax.axis_index("s")
    rpt = (_E // _C1) // _NT
    nsb = rpt // _SBR

    @pl.when(c == 0)
    def _():
        _seg_body(hA_hbm, ahA_hbm, send2d_hbm, recv2d_hbm, zeros_hbm,
                  sidx, ridx, rows_v, acc, gsem, ssem, s, _K1, nsb,
                  3, _C1)

    @pl.when(c == 1)
    def _():
        _seg_body(hB_hbm, ahB_hbm, send2d_hbm, recv2d_hbm, zeros_hbm,
                  sidx, ridx, rows_v, acc, gsem, ssem, s, _K1, nsb,
                  3, _C1)


_BR = 2048
_BRP = 2000


def _prep_body(nodes_ref, ds_ref, outa_ref, outb_ref):
    ns = lax.rsqrt(jnp.maximum(ds_ref[:, 0], 1.0))
    xn = nodes_ref[...] * ns[:, None]
    rows = xn.shape[0]
    pad7 = jnp.zeros((rows, _HALFW - 65), _f32)
    pad8 = jnp.zeros((rows, _HALFW - 64), _f32)
    outa_ref[...] = jnp.concatenate([xn[:, :64], ns[:, None], pad7], axis=1)
    outb_ref[...] = jnp.concatenate([xn[:, 64:], pad8], axis=1)


def _prep(nodes, ds16):
    return pl.pallas_call(
        _prep_body,
        grid=(_N // _BRP,),
        in_specs=[pl.BlockSpec((_BRP, _D), lambda i: (i, 0)),
                  pl.BlockSpec((_BRP, _DEGW), lambda i: (i, 0))],
        out_specs=[pl.BlockSpec((_BRP, _HALFW), lambda i: (i, 0)),
                   pl.BlockSpec((_BRP, _HALFW), lambda i: (i, 0))],
        out_shape=[jax.ShapeDtypeStruct((_N, _HALFW), _f32),
                   jax.ShapeDtypeStruct((_N, _HALFW), _f32)],
    )(nodes, ds16)


def _layer0_dense_body(axA_ref, axB_ref, dr_ref, ds_ref, w0a_ref, w0b_ref,
                       outa_ref, outb_ref):
    t = (jnp.dot(axA_ref[...], w0a_ref[...], preferred_element_type=_f32)
         + jnp.dot(axB_ref[...], w0b_ref[...], preferred_element_type=_f32))
    nr = lax.rsqrt(jnp.maximum(dr_ref[:, 0], 1.0))
    ns = lax.rsqrt(jnp.maximum(ds_ref[:, 0], 1.0))
    h0n = jnp.maximum(t * nr[:, None], 0.0) * ns[:, None]
    outa_ref[...] = h0n[:, :_D]
    outb_ref[...] = h0n[:, _D:]


def _layer0_dense(axA, axB, dr16, ds16, w0a, w0b):
    return pl.pallas_call(
        _layer0_dense_body,
        grid=(_NP // _BR,),
        in_specs=[pl.BlockSpec((_BR, _HALFW), lambda i: (i, 0)),
                  pl.BlockSpec((_BR, _HALFW), lambda i: (i, 0)),
                  pl.BlockSpec((_BR, _DEGW), lambda i: (i, 0)),
                  pl.BlockSpec((_BR, _DEGW), lambda i: (i, 0)),
                  pl.BlockSpec((_HALFW, _H), lambda i: (0, 0)),
                  pl.BlockSpec((_HALFW, _H), lambda i: (0, 0))],
        out_specs=[pl.BlockSpec((_BR, _D), lambda i: (i, 0)),
                   pl.BlockSpec((_BR, _D), lambda i: (i, 0))],
        out_shape=[jax.ShapeDtypeStruct((_NP, _D), _f32),
                   jax.ShapeDtypeStruct((_NP, _D), _f32)],
    )(axA, axB, dr16, ds16, w0a, w0b)


def _layer1_dense_body(ahA_ref, ahB_ref, axA_ref, axB_ref, dr_ref,
                       w1hi_ref, w1lo_ref, w1ba_ref, w1bb_ref,
                       w2_ref, b2_ref, inv_ref, out_ref, acc_ref):
    i = pl.program_id(0)
    g = (jnp.dot(ahA_ref[...], w1hi_ref[...], preferred_element_type=_f32)
         + jnp.dot(ahB_ref[...], w1lo_ref[...], preferred_element_type=_f32)
         + jnp.dot(axA_ref[...], w1ba_ref[...], preferred_element_type=_f32)
         + jnp.dot(axB_ref[...], w1bb_ref[...], preferred_element_type=_f32))
    nr = lax.rsqrt(jnp.maximum(dr_ref[:, 0], 1.0))
    h1 = jnp.maximum(g * nr[:, None], 0.0)
    psum = jnp.sum(h1, axis=0, keepdims=True)
    acc_ref[...] = jnp.where(i == 0, psum, acc_ref[...] + psum)

    @pl.when(i == pl.num_programs(0) - 1)
    def _():
        pooled = acc_ref[...] * inv_ref[0, 0]
        out_ref[...] = (jnp.dot(pooled, w2_ref[...], preferred_element_type=_f32)
                        + b2_ref[...])


def _layer1_dense(ahA, ahB, axA, axB, dr16, w1hi, w1lo, w1ba, w1bb,
                  w2, b2, inv):
    return pl.pallas_call(
        _layer1_dense_body,
        grid=(_NP // _BR,),
        in_specs=[pl.BlockSpec((_BR, _D), lambda i: (i, 0)),
                  pl.BlockSpec((_BR, _D), lambda i: (i, 0)),
                  pl.BlockSpec((_BR, _HALFW), lambda i: (i, 0)),
                  pl.BlockSpec((_BR, _HALFW), lambda i: (i, 0)),
                  pl.BlockSpec((_BR, _DEGW), lambda i: (i, 0)),
                  pl.BlockSpec((_D, _H), lambda i: (0, 0)),
                  pl.BlockSpec((_D, _H), lambda i: (0, 0)),
                  pl.BlockSpec((_HALFW, _H), lambda i: (0, 0)),
                  pl.BlockSpec((_HALFW, _H), lambda i: (0, 0)),
                  pl.BlockSpec((_H, _OUT), lambda i: (0, 0)),
                  pl.BlockSpec((1, _OUT), lambda i: (0, 0)),
                  pl.BlockSpec((1, 1), lambda i: (0, 0))],
        out_specs=pl.BlockSpec((1, _OUT), lambda i: (0, 0)),
        out_shape=jax.ShapeDtypeStruct((1, _OUT), _f32),
        scratch_shapes=[pltpu.VMEM((1, _H), _f32)],
    )(ahA, ahB, axA, axB, dr16, w1hi, w1lo, w1ba, w1bb, w2, b2, inv)


def kernel(nodes, senders, receivers, n_node, W0, b0, W1, b1, W2, b2):
    ones16 = jnp.ones((_C, _DEGW), _f32)
    zeros_deg = jnp.zeros((_NPT, _DEGW), _f32)
    zeros_half = jnp.zeros((_NPT, _HALFW), _f32)
    zeros_d = jnp.zeros((_NPT, _D), _f32)
    send2d = senders.reshape(_E // _C, _C)
    recv2d = receivers.reshape(_E // _C, _C)
    send2d1 = senders.reshape(_E // _C1, _C1)
    recv2d1 = receivers.reshape(_E // _C1, _C1)

    ds16, dr16 = _deg_kernel(send2d, recv2d, ones16, zeros_deg)
    xnA, xnB = _prep(nodes, ds16)
    axA, axB = _l0_kernel(xnA, xnB, send2d, recv2d, zeros_half)

    pad7 = jnp.zeros((_HALFW - 65, _H), _f32)
    pad8 = jnp.zeros((_HALFW - 64, _H), _f32)
    w0a = jnp.concatenate([W0[:64], b0[None, :], pad7], axis=0)
    w0b = jnp.concatenate([W0[64:], pad8], axis=0)
    hA, hB = _layer0_dense(axA, axB, dr16, ds16, w0a, w0b)

    ahA, ahB = _l1_kernel(hA, hB, send2d1, recv2d1, zeros_d)

    w1hi = W1[:_D]
    w1lo = W1[_D:_H]
    w1ba = jnp.concatenate([W1[_H:_H + 64], b1[None, :], pad7], axis=0)
    w1bb = jnp.concatenate([W1[_H + 64:], pad8], axis=0)
    inv = (1.0 / jnp.maximum(n_node.astype(_f32), 1.0)).reshape(1, 1)
    out = _layer1_dense(ahA, ahB, axA, axB, dr16, w1hi, w1lo, w1ba, w1bb,
                        W2, b2.reshape(1, _OUT), inv)
    return out.reshape(_OUT)


# speedup vs baseline: 1.1066x; 1.0086x over previous
import functools

import jax
import jax.numpy as jnp
from jax import lax
from jax.experimental import pallas as pl
from jax.experimental.pallas import tpu as pltpu
from jax.experimental.pallas import tpu_sc as plsc

_N = 10000
_E = 320000
_D = 128
_H = 256
_OUT = 128

_NT = 16
_NC = 2
_C = 80
_K0 = 12
_C1 = 80
_K1 = 4
_SBR = 25
_HALFW = 72
_DEGW = 8
_NP = 10240
_NPT = _NP // _NT

_mesh = plsc.VectorSubcoreMesh(core_axis_name="c", subcore_axis_name="s")
_f32 = jnp.float32


@functools.partial(
    pl.kernel,
    out_type=[jax.ShapeDtypeStruct((_NP, _DEGW), _f32),
              jax.ShapeDtypeStruct((_NP, _DEGW), _f32)],
    mesh=_mesh,
    scratch_types=[pltpu.VMEM((_C, _DEGW), _f32),
                   pltpu.VMEM((_SBR, _C), jnp.int32),
                   pltpu.VMEM_SHARED((_NP, _DEGW), _f32),
                   pltpu.SemaphoreType.DMA],
    compiler_params=pltpu.CompilerParams(use_tc_tiling_on_sc=False),
)
def _deg_kernel(send2d_hbm, recv2d_hbm, ones_hbm, zeros_hbm, ds_hbm, dr_hbm,
                ones_v, idx_v, acc, sem):
    c = lax.axis_index("c")
    s = lax.axis_index("s")
    pltpu.sync_copy(zeros_hbm, acc.at[pl.ds(s * _NPT, _NPT)])
    pltpu.sync_copy(ones_hbm, ones_v)
    plsc.subcore_barrier()

    rpt = (_E // _C) // _NT
    nsb = rpt // _SBR

    def hist(idx_hbm, out_hbm):
        def superblock(sb, carry):
            r0 = s * rpt + sb * _SBR
            pltpu.sync_copy(idx_hbm.at[pl.ds(r0, _SBR)], idx_v)
            adds = [pltpu.async_copy(ones_v, acc.at[idx_v.at[j]], sem,
                                     add=True)
                    for j in range(_SBR)]
            for t in adds:
                t.wait()
            return carry
        lax.fori_loop(0, nsb, superblock, 0)
        plsc.subcore_barrier()
        pltpu.sync_copy(acc.at[pl.ds(s * _NPT, _NPT)],
                        out_hbm.at[pl.ds(s * _NPT, _NPT)])

    @pl.when(c == 0)
    def _():
        hist(send2d_hbm, ds_hbm)

    @pl.when(c == 1)
    def _():
        hist(recv2d_hbm, dr_hbm)


def _seg_body(table_hbm, out_hbm, send2d, recv2d, zeros_hbm,
              sidx, ridx, rows_v, acc, gsems, ssems, s, k, nsb, lag, csz):
    pltpu.sync_copy(zeros_hbm, acc.at[pl.ds(s * _NPT, _NPT)])
    plsc.subcore_barrier()
    row_base = s * (nsb * _SBR)

    def superblock(sb, carry):
        r0 = row_base + sb * _SBR
        pltpu.sync_copy(send2d.at[pl.ds(r0, _SBR)], sidx)
        pltpu.sync_copy(recv2d.at[pl.ds(r0, _SBR)], ridx)

        gdesc = [None] * _SBR
        sdesc = [None] * _SBR

        def fire_scatter(t):
            slot = t % k
            gdesc[t].wait()
            sdesc[t] = pltpu.async_copy(rows_v.at[pl.ds(slot * csz, csz)],
                                        acc.at[ridx.at[t]], ssems.at[slot],
                                        add=True)

        for j in range(_SBR):
            slot = j % k
            if j >= k:
                sdesc[j - k].wait()
            gdesc[j] = pltpu.async_copy(table_hbm.at[sidx.at[j]],
                                        rows_v.at[pl.ds(slot * csz, csz)],
                                        gsems.at[slot])
            if j >= lag:
                fire_scatter(j - lag)
        for t in range(_SBR - lag, _SBR):
            fire_scatter(t)
        for t in range(_SBR - k, _SBR):
            sdesc[t].wait()
        return carry

    lax.fori_loop(0, nsb, superblock, 0)
    plsc.subcore_barrier()
    pltpu.sync_copy(acc.at[pl.ds(s * _NPT, _NPT)],
                    out_hbm.at[pl.ds(s * _NPT, _NPT)])


@functools.partial(
    pl.kernel,
    out_type=[jax.ShapeDtypeStruct((_NP, _HALFW), _f32),
              jax.ShapeDtypeStruct((_NP, _HALFW), _f32)],
    mesh=_mesh,
    scratch_types=[pltpu.VMEM((_SBR, _C), jnp.int32),
                   pltpu.VMEM((_SBR, _C), jnp.int32),
                   pltpu.VMEM((_K0 * _C, _HALFW), _f32),
                   pltpu.VMEM_SHARED((_NP, _HALFW), _f32),
                   pltpu.SemaphoreType.DMA((_K0,)),
                   pltpu.SemaphoreType.DMA((_K0,))],
    compiler_params=pltpu.CompilerParams(use_tc_tiling_on_sc=False),
)
def _l0_kernel(xnA_hbm, xnB_hbm, send2d_hbm, recv2d_hbm, zeros_hbm,
               axA_hbm, axB_hbm, sidx, ridx, rows_v, acc, gsem, ssem):
    c = lax.axis_index("c")
    s = lax.axis_index("s")
    rpt = (_E // _C) // _NT
    nsb = rpt // _SBR

    @pl.when(c == 0)
    def _():
        _seg_body(xnA_hbm, axA_hbm, send2d_hbm, recv2d_hbm, zeros_hbm,
                  sidx, ridx, rows_v, acc, gsem, ssem, s, _K0, nsb,
                  8, _C)

    @pl.when(c == 1)
    def _():
        _seg_body(xnB_hbm, axB_hbm, send2d_hbm, recv2d_hbm, zeros_hbm,
                  sidx, ridx, rows_v, acc, gsem, ssem, s, _K0, nsb,
                  8, _C)


@functools.partial(
    pl.kernel,
    out_type=[jax.ShapeDtypeStruct((_NP, _D), _f32),
              jax.ShapeDtypeStruct((_NP, _D), _f32)],
    mesh=_mesh,
    scratch_types=[pltpu.VMEM((_SBR, _C1), jnp.int32),
                   pltpu.VMEM((_SBR, _C1), jnp.int32),
                   pltpu.VMEM((_K1 * _C1, _D), _f32),
                   pltpu.VMEM_SHARED((_NP, _D), _f32),
                   pltpu.SemaphoreType.DMA((_K1,)),
                   pltpu.SemaphoreType.DMA((_K1,))],
    compiler_params=pltpu.CompilerParams(use_tc_tiling_on_sc=False),
)
def _l1_kernel(hA_hbm, hB_hbm, send2d_hbm, recv2d_hbm, zeros_hbm,
               ahA_hbm, ahB_hbm, sidx, ridx, rows_v, acc, gsem, ssem):
    c = lax.axis_index("c")
    s = lax.axis_index("s")
    rpt = (_E // _C1) // _NT
    nsb = rpt // _SBR

    @pl.when(c == 0)
    def _():
        _seg_body(hA_hbm, ahA_hbm, send2d_hbm, recv2d_hbm, zeros_hbm,
                  sidx, ridx, rows_v, acc, gsem, ssem, s, _K1, nsb,
                  3, _C1)

    @pl.when(c == 1)
    def _():
        _seg_body(hB_hbm, ahB_hbm, send2d_hbm, recv2d_hbm, zeros_hbm,
                  sidx, ridx, rows_v, acc, gsem, ssem, s, _K1, nsb,
                  3, _C1)


_BR = 2048
_BRP = 2000


def _prep_body(nodes_ref, ds_ref, outa_ref, outb_ref):
    ns = lax.rsqrt(jnp.maximum(ds_ref[:, 0], 1.0))
    xn = nodes_ref[...] * ns[:, None]
    rows = xn.shape[0]
    pad7 = jnp.zeros((rows, _HALFW - 65), _f32)
    pad8 = jnp.zeros((rows, _HALFW - 64), _f32)
    outa_ref[...] = jnp.concatenate([xn[:, :64], ns[:, None], pad7], axis=1)
    outb_ref[...] = jnp.concatenate([xn[:, 64:], pad8], axis=1)


def _prep(nodes, ds16):
    return pl.pallas_call(
        _prep_body,
        grid=(_N // _BRP,),
        in_specs=[pl.BlockSpec((_BRP, _D), lambda i: (i, 0)),
                  pl.BlockSpec((_BRP, _DEGW), lambda i: (i, 0))],
        out_specs=[pl.BlockSpec((_BRP, _HALFW), lambda i: (i, 0)),
                   pl.BlockSpec((_BRP, _HALFW), lambda i: (i, 0))],
        out_shape=[jax.ShapeDtypeStruct((_N, _HALFW), _f32),
                   jax.ShapeDtypeStruct((_N, _HALFW), _f32)],
    )(nodes, ds16)


def _layer0_dense_body(axA_ref, axB_ref, dr_ref, ds_ref, w0a_ref, w0b_ref,
                       outa_ref, outb_ref):
    t = (jnp.dot(axA_ref[...], w0a_ref[...], preferred_element_type=_f32)
         + jnp.dot(axB_ref[...], w0b_ref[...], preferred_element_type=_f32))
    nr = lax.rsqrt(jnp.maximum(dr_ref[:, 0], 1.0))
    ns = lax.rsqrt(jnp.maximum(ds_ref[:, 0], 1.0))
    h0n = jnp.maximum(t * nr[:, None], 0.0) * ns[:, None]
    outa_ref[...] = h0n[:, :_D]
    outb_ref[...] = h0n[:, _D:]


def _layer0_dense(axA, axB, dr16, ds16, w0a, w0b):
    return pl.pallas_call(
        _layer0_dense_body,
        grid=(_NP // _BR,),
        in_specs=[pl.BlockSpec((_BR, _HALFW), lambda i: (i, 0)),
                  pl.BlockSpec((_BR, _HALFW), lambda i: (i, 0)),
                  pl.BlockSpec((_BR, _DEGW), lambda i: (i, 0)),
                  pl.BlockSpec((_BR, _DEGW), lambda i: (i, 0)),
                  pl.BlockSpec((_HALFW, _H), lambda i: (0, 0)),
                  pl.BlockSpec((_HALFW, _H), lambda i: (0, 0))],
        out_specs=[pl.BlockSpec((_BR, _D), lambda i: (i, 0)),
                   pl.BlockSpec((_BR, _D), lambda i: (i, 0))],
        out_shape=[jax.ShapeDtypeStruct((_NP, _D), _f32),
                   jax.ShapeDtypeStruct((_NP, _D), _f32)],
    )(axA, axB, dr16, ds16, w0a, w0b)


def _layer1_dense_body(ahA_ref, ahB_ref, axA_ref, axB_ref, dr_ref,
                       w1hi_ref, w1lo_ref, w1ba_ref, w1bb_ref,
                       w2_ref, b2_ref, inv_ref, out_ref, acc_ref):
    i = pl.program_id(0)
    g = (jnp.dot(ahA_ref[...], w1hi_ref[...], preferred_element_type=_f32)
         + jnp.dot(ahB_ref[...], w1lo_ref[...], preferred_element_type=_f32)
         + jnp.dot(axA_ref[...], w1ba_ref[...], preferred_element_type=_f32)
         + jnp.dot(axB_ref[...], w1bb_ref[...], preferred_element_type=_f32))
    nr = lax.rsqrt(jnp.maximum(dr_ref[:, 0], 1.0))
    h1 = jnp.maximum(g * nr[:, None], 0.0)
    psum = jnp.sum(h1, axis=0, keepdims=True)
    acc_ref[...] = jnp.where(i == 0, psum, acc_ref[...] + psum)

    @pl.when(i == pl.num_programs(0) - 1)
    def _():
        pooled = acc_ref[...] * inv_ref[0, 0]
        out_ref[...] = (jnp.dot(pooled, w2_ref[...], preferred_element_type=_f32)
                        + b2_ref[...])


def _layer1_dense(ahA, ahB, axA, axB, dr16, w1hi, w1lo, w1ba, w1bb,
                  w2, b2, inv):
    return pl.pallas_call(
        _layer1_dense_body,
        grid=(_NP // _BR,),
        in_specs=[pl.BlockSpec((_BR, _D), lambda i: (i, 0)),
                  pl.BlockSpec((_BR, _D), lambda i: (i, 0)),
                  pl.BlockSpec((_BR, _HALFW), lambda i: (i, 0)),
                  pl.BlockSpec((_BR, _HALFW), lambda i: (i, 0)),
                  pl.BlockSpec((_BR, _DEGW), lambda i: (i, 0)),
                  pl.BlockSpec((_D, _H), lambda i: (0, 0)),
                  pl.BlockSpec((_D, _H), lambda i: (0, 0)),
                  pl.BlockSpec((_HALFW, _H), lambda i: (0, 0)),
                  pl.BlockSpec((_HALFW, _H), lambda i: (0, 0)),
                  pl.BlockSpec((_H, _OUT), lambda i: (0, 0)),
                  pl.BlockSpec((1, _OUT), lambda i: (0, 0)),
                  pl.BlockSpec((1, 1), lambda i: (0, 0))],
        out_specs=pl.BlockSpec((1, _OUT), lambda i: (0, 0)),
        out_shape=jax.ShapeDtypeStruct((1, _OUT), _f32),
        scratch_shapes=[pltpu.VMEM((1, _H), _f32)],
    )(ahA, ahB, axA, axB, dr16, w1hi, w1lo, w1ba, w1bb, w2, b2, inv)


def kernel(nodes, senders, receivers, n_node, W0, b0, W1, b1, W2, b2):
    ones16 = jnp.ones((_C, _DEGW), _f32)
    zeros_deg = jnp.zeros((_NPT, _DEGW), _f32)
    zeros_half = jnp.zeros((_NPT, _HALFW), _f32)
    zeros_d = jnp.zeros((_NPT, _D), _f32)
    send2d = senders.reshape(_E // _C, _C)
    recv2d = receivers.reshape(_E // _C, _C)
    send2d1 = senders.reshape(_E // _C1, _C1)
    recv2d1 = receivers.reshape(_E // _C1, _C1)

    ds16, dr16 = _deg_kernel(send2d, recv2d, ones16, zeros_deg)
    xnA, xnB = _prep(nodes, ds16)
    axA, axB = _l0_kernel(xnA, xnB, send2d, recv2d, zeros_half)

    pad7 = jnp.zeros((_HALFW - 65, _H), _f32)
    pad8 = jnp.zeros((_HALFW - 64, _H), _f32)
    w0a = jnp.concatenate([W0[:64], b0[None, :], pad7], axis=0)
    w0b = jnp.concatenate([W0[64:], pad8], axis=0)
    hA, hB = _layer0_dense(axA, axB, dr16, ds16, w0a, w0b)

    ahA, ahB = _l1_kernel(hA, hB, send2d1, recv2d1, zeros_d)

    w1hi = W1[:_D]
    w1lo = W1[_D:_H]
    w1ba = jnp.concatenate([W1[_H:_H + 64], b1[None, :], pad7], axis=0)
    w1bb = jnp.concatenate([W1[_H + 64:], pad8], axis=0)
    inv = (1.0 / jnp.maximum(n_node.astype(_f32), 1.0)).reshape(1, 1)
    out = _layer1_dense(ahA, ahB, axA, axB, dr16, w1hi, w1lo, w1ba, w1bb,
                        W2, b2.reshape(1, _OUT), inv)
    return out.reshape(_OUT)


# speedup vs baseline: 1.1416x; 1.0316x over previous
import functools

import jax
import jax.numpy as jnp
from jax import lax
from jax.experimental import pallas as pl
from jax.experimental.pallas import tpu as pltpu
from jax.experimental.pallas import tpu_sc as plsc

_N = 10000
_E = 320000
_D = 128
_H = 256
_OUT = 128

_NT = 16
_NC = 2
_C = 80
_K0 = 12
_C1 = 80
_K1 = 4
_SBR = 25
_SBR1 = 50
_HALFW = 72
_DEGW = 8
_NP = 10240
_NPT = _NP // _NT

_mesh = plsc.VectorSubcoreMesh(core_axis_name="c", subcore_axis_name="s")
_f32 = jnp.float32


@functools.partial(
    pl.kernel,
    out_type=[jax.ShapeDtypeStruct((_NP, _DEGW), _f32),
              jax.ShapeDtypeStruct((_NP, _DEGW), _f32)],
    mesh=_mesh,
    scratch_types=[pltpu.VMEM((_C, _DEGW), _f32),
                   pltpu.VMEM((_SBR, _C), jnp.int32),
                   pltpu.VMEM_SHARED((_NP, _DEGW), _f32),
                   pltpu.SemaphoreType.DMA],
    compiler_params=pltpu.CompilerParams(use_tc_tiling_on_sc=False),
)
def _deg_kernel(send2d_hbm, recv2d_hbm, ones_hbm, zeros_hbm, ds_hbm, dr_hbm,
                ones_v, idx_v, acc, sem):
    c = lax.axis_index("c")
    s = lax.axis_index("s")
    pltpu.sync_copy(zeros_hbm, acc.at[pl.ds(s * _NPT, _NPT)])
    pltpu.sync_copy(ones_hbm, ones_v)
    plsc.subcore_barrier()

    rpt = (_E // _C) // _NT
    nsb = rpt // _SBR

    def hist(idx_hbm, out_hbm):
        def superblock(sb, carry):
            r0 = s * rpt + sb * _SBR
            pltpu.sync_copy(idx_hbm.at[pl.ds(r0, _SBR)], idx_v)
            adds = [pltpu.async_copy(ones_v, acc.at[idx_v.at[j]], sem,
                                     add=True)
                    for j in range(_SBR)]
            for t in adds:
                t.wait()
            return carry
        lax.fori_loop(0, nsb, superblock, 0)
        plsc.subcore_barrier()
        pltpu.sync_copy(acc.at[pl.ds(s * _NPT, _NPT)],
                        out_hbm.at[pl.ds(s * _NPT, _NPT)])

    @pl.when(c == 0)
    def _():
        hist(send2d_hbm, ds_hbm)

    @pl.when(c == 1)
    def _():
        hist(recv2d_hbm, dr_hbm)


def _seg_body(table_hbm, out_hbm, send2d, recv2d, zeros_hbm,
              sidx, ridx, rows_v, acc, gsems, ssems, s, k, nsb, lag, csz, sbr):
    pltpu.sync_copy(zeros_hbm, acc.at[pl.ds(s * _NPT, _NPT)])
    plsc.subcore_barrier()
    row_base = s * (nsb * sbr)

    def superblock(sb, carry):
        r0 = row_base + sb * sbr
        pltpu.sync_copy(send2d.at[pl.ds(r0, sbr)], sidx)
        pltpu.sync_copy(recv2d.at[pl.ds(r0, sbr)], ridx)

        gdesc = [None] * sbr
        sdesc = [None] * sbr

        def fire_scatter(t):
            slot = t % k
            gdesc[t].wait()
            sdesc[t] = pltpu.async_copy(rows_v.at[pl.ds(slot * csz, csz)],
                                        acc.at[ridx.at[t]], ssems.at[slot],
                                        add=True)

        for j in range(sbr):
            slot = j % k
            if j >= k:
                sdesc[j - k].wait()
            gdesc[j] = pltpu.async_copy(table_hbm.at[sidx.at[j]],
                                        rows_v.at[pl.ds(slot * csz, csz)],
                                        gsems.at[slot])
            if j >= lag:
                fire_scatter(j - lag)
        for t in range(sbr - lag, sbr):
            fire_scatter(t)
        for t in range(sbr - k, sbr):
            sdesc[t].wait()
        return carry

    lax.fori_loop(0, nsb, superblock, 0)
    plsc.subcore_barrier()
    pltpu.sync_copy(acc.at[pl.ds(s * _NPT, _NPT)],
                    out_hbm.at[pl.ds(s * _NPT, _NPT)])


@functools.partial(
    pl.kernel,
    out_type=[jax.ShapeDtypeStruct((_NP, _HALFW), _f32),
              jax.ShapeDtypeStruct((_NP, _HALFW), _f32)],
    mesh=_mesh,
    scratch_types=[pltpu.VMEM((_SBR, _C), jnp.int32),
                   pltpu.VMEM((_SBR, _C), jnp.int32),
                   pltpu.VMEM((_K0 * _C, _HALFW), _f32),
                   pltpu.VMEM_SHARED((_NP, _HALFW), _f32),
                   pltpu.SemaphoreType.DMA((_K0,)),
                   pltpu.SemaphoreType.DMA((_K0,))],
    compiler_params=pltpu.CompilerParams(use_tc_tiling_on_sc=False),
)
def _l0_kernel(xnA_hbm, xnB_hbm, send2d_hbm, recv2d_hbm, zeros_hbm,
               axA_hbm, axB_hbm, sidx, ridx, rows_v, acc, gsem, ssem):
    c = lax.axis_index("c")
    s = lax.axis_index("s")
    rpt = (_E // _C) // _NT
    nsb = rpt // _SBR

    @pl.when(c == 0)
    def _():
        _seg_body(xnA_hbm, axA_hbm, send2d_hbm, recv2d_hbm, zeros_hbm,
                  sidx, ridx, rows_v, acc, gsem, ssem, s, _K0, nsb,
                  8, _C, _SBR)

    @pl.when(c == 1)
    def _():
        _seg_body(xnB_hbm, axB_hbm, send2d_hbm, recv2d_hbm, zeros_hbm,
                  sidx, ridx, rows_v, acc, gsem, ssem, s, _K0, nsb,
                  8, _C, _SBR)


@functools.partial(
    pl.kernel,
    out_type=[jax.ShapeDtypeStruct((_NP, _D), _f32),
              jax.ShapeDtypeStruct((_NP, _D), _f32)],
    mesh=_mesh,
    scratch_types=[pltpu.VMEM((_SBR1, _C1), jnp.int32),
                   pltpu.VMEM((_SBR1, _C1), jnp.int32),
                   pltpu.VMEM((_K1 * _C1, _D), _f32),
                   pltpu.VMEM_SHARED((_NP, _D), _f32),
                   pltpu.SemaphoreType.DMA((_K1,)),
                   pltpu.SemaphoreType.DMA((_K1,))],
    compiler_params=pltpu.CompilerParams(use_tc_tiling_on_sc=False),
)
def _l1_kernel(hA_hbm, hB_hbm, send2d_hbm, recv2d_hbm, zeros_hbm,
               ahA_hbm, ahB_hbm, sidx, ridx, rows_v, acc, gsem, ssem):
    c = lax.axis_index("c")
    s = lax.axis_index("s")
    rpt = (_E // _C1) // _NT
    nsb = rpt // _SBR1

    @pl.when(c == 0)
    def _():
        _seg_body(hA_hbm, ahA_hbm, send2d_hbm, recv2d_hbm, zeros_hbm,
                  sidx, ridx, rows_v, acc, gsem, ssem, s, _K1, nsb,
                  3, _C1, _SBR1)

    @pl.when(c == 1)
    def _():
        _seg_body(hB_hbm, ahB_hbm, send2d_hbm, recv2d_hbm, zeros_hbm,
                  sidx, ridx, rows_v, acc, gsem, ssem, s, _K1, nsb,
                  3, _C1, _SBR1)


_BR = 2048
_BRP = 2000


def _prep_body(nodes_ref, ds_ref, outa_ref, outb_ref):
    ns = lax.rsqrt(jnp.maximum(ds_ref[:, 0], 1.0))
    xn = nodes_ref[...] * ns[:, None]
    rows = xn.shape[0]
    pad7 = jnp.zeros((rows, _HALFW - 65), _f32)
    pad8 = jnp.zeros((rows, _HALFW - 64), _f32)
    outa_ref[...] = jnp.concatenate([xn[:, :64], ns[:, None], pad7], axis=1)
    outb_ref[...] = jnp.concatenate([xn[:, 64:], pad8], axis=1)


def _prep(nodes, ds16):
    return pl.pallas_call(
        _prep_body,
        grid=(_N // _BRP,),
        in_specs=[pl.BlockSpec((_BRP, _D), lambda i: (i, 0)),
                  pl.BlockSpec((_BRP, _DEGW), lambda i: (i, 0))],
        out_specs=[pl.BlockSpec((_BRP, _HALFW), lambda i: (i, 0)),
                   pl.BlockSpec((_BRP, _HALFW), lambda i: (i, 0))],
        out_shape=[jax.ShapeDtypeStruct((_N, _HALFW), _f32),
                   jax.ShapeDtypeStruct((_N, _HALFW), _f32)],
    )(nodes, ds16)


def _layer0_dense_body(axA_ref, axB_ref, dr_ref, ds_ref, w0a_ref, w0b_ref,
                       outa_ref, outb_ref):
    t = (jnp.dot(axA_ref[...], w0a_ref[...], preferred_element_type=_f32)
         + jnp.dot(axB_ref[...], w0b_ref[...], preferred_element_type=_f32))
    nr = lax.rsqrt(jnp.maximum(dr_ref[:, 0], 1.0))
    ns = lax.rsqrt(jnp.maximum(ds_ref[:, 0], 1.0))
    h0n = jnp.maximum(t * nr[:, None], 0.0) * ns[:, None]
    outa_ref[...] = h0n[:, :_D]
    outb_ref[...] = h0n[:, _D:]


def _layer0_dense(axA, axB, dr16, ds16, w0a, w0b):
    return pl.pallas_call(
        _layer0_dense_body,
        grid=(_NP // _BR,),
        in_specs=[pl.BlockSpec((_BR, _HALFW), lambda i: (i, 0)),
                  pl.BlockSpec((_BR, _HALFW), lambda i: (i, 0)),
                  pl.BlockSpec((_BR, _DEGW), lambda i: (i, 0)),
                  pl.BlockSpec((_BR, _DEGW), lambda i: (i, 0)),
                  pl.BlockSpec((_HALFW, _H), lambda i: (0, 0)),
                  pl.BlockSpec((_HALFW, _H), lambda i: (0, 0))],
        out_specs=[pl.BlockSpec((_BR, _D), lambda i: (i, 0)),
                   pl.BlockSpec((_BR, _D), lambda i: (i, 0))],
        out_shape=[jax.ShapeDtypeStruct((_NP, _D), _f32),
                   jax.ShapeDtypeStruct((_NP, _D), _f32)],
    )(axA, axB, dr16, ds16, w0a, w0b)


def _layer1_dense_body(ahA_ref, ahB_ref, axA_ref, axB_ref, dr_ref,
                       w1hi_ref, w1lo_ref, w1ba_ref, w1bb_ref,
                       w2_ref, b2_ref, inv_ref, out_ref, acc_ref):
    i = pl.program_id(0)
    g = (jnp.dot(ahA_ref[...], w1hi_ref[...], preferred_element_type=_f32)
         + jnp.dot(ahB_ref[...], w1lo_ref[...], preferred_element_type=_f32)
         + jnp.dot(axA_ref[...], w1ba_ref[...], preferred_element_type=_f32)
         + jnp.dot(axB_ref[...], w1bb_ref[...], preferred_element_type=_f32))
    nr = lax.rsqrt(jnp.maximum(dr_ref[:, 0], 1.0))
    h1 = jnp.maximum(g * nr[:, None], 0.0)
    psum = jnp.sum(h1, axis=0, keepdims=True)
    acc_ref[...] = jnp.where(i == 0, psum, acc_ref[...] + psum)

    @pl.when(i == pl.num_programs(0) - 1)
    def _():
        pooled = acc_ref[...] * inv_ref[0, 0]
        out_ref[...] = (jnp.dot(pooled, w2_ref[...], preferred_element_type=_f32)
                        + b2_ref[...])


def _layer1_dense(ahA, ahB, axA, axB, dr16, w1hi, w1lo, w1ba, w1bb,
                  w2, b2, inv):
    return pl.pallas_call(
        _layer1_dense_body,
        grid=(_NP // _BR,),
        in_specs=[pl.BlockSpec((_BR, _D), lambda i: (i, 0)),
                  pl.BlockSpec((_BR, _D), lambda i: (i, 0)),
                  pl.BlockSpec((_BR, _HALFW), lambda i: (i, 0)),
                  pl.BlockSpec((_BR, _HALFW), lambda i: (i, 0)),
                  pl.BlockSpec((_BR, _DEGW), lambda i: (i, 0)),
                  pl.BlockSpec((_D, _H), lambda i: (0, 0)),
                  pl.BlockSpec((_D, _H), lambda i: (0, 0)),
                  pl.BlockSpec((_HALFW, _H), lambda i: (0, 0)),
                  pl.BlockSpec((_HALFW, _H), lambda i: (0, 0)),
                  pl.BlockSpec((_H, _OUT), lambda i: (0, 0)),
                  pl.BlockSpec((1, _OUT), lambda i: (0, 0)),
                  pl.BlockSpec((1, 1), lambda i: (0, 0))],
        out_specs=pl.BlockSpec((1, _OUT), lambda i: (0, 0)),
        out_shape=jax.ShapeDtypeStruct((1, _OUT), _f32),
        scratch_shapes=[pltpu.VMEM((1, _H), _f32)],
    )(ahA, ahB, axA, axB, dr16, w1hi, w1lo, w1ba, w1bb, w2, b2, inv)


def kernel(nodes, senders, receivers, n_node, W0, b0, W1, b1, W2, b2):
    ones16 = jnp.ones((_C, _DEGW), _f32)
    zeros_deg = jnp.zeros((_NPT, _DEGW), _f32)
    zeros_half = jnp.zeros((_NPT, _HALFW), _f32)
    zeros_d = jnp.zeros((_NPT, _D), _f32)
    send2d = senders.reshape(_E // _C, _C)
    recv2d = receivers.reshape(_E // _C, _C)
    send2d1 = senders.reshape(_E // _C1, _C1)
    recv2d1 = receivers.reshape(_E // _C1, _C1)

    ds16, dr16 = _deg_kernel(send2d, recv2d, ones16, zeros_deg)
    xnA, xnB = _prep(nodes, ds16)
    axA, axB = _l0_kernel(xnA, xnB, send2d, recv2d, zeros_half)

    pad7 = jnp.zeros((_HALFW - 65, _H), _f32)
    pad8 = jnp.zeros((_HALFW - 64, _H), _f32)
    w0a = jnp.concatenate([W0[:64], b0[None, :], pad7], axis=0)
    w0b = jnp.concatenate([W0[64:], pad8], axis=0)
    hA, hB = _layer0_dense(axA, axB, dr16, ds16, w0a, w0b)

    ahA, ahB = _l1_kernel(hA, hB, send2d1, recv2d1, zeros_d)

    w1hi = W1[:_D]
    w1lo = W1[_D:_H]
    w1ba = jnp.concatenate([W1[_H:_H + 64], b1[None, :], pad7], axis=0)
    w1bb = jnp.concatenate([W1[_H + 64:], pad8], axis=0)
    inv = (1.0 / jnp.maximum(n_node.astype(_f32), 1.0)).reshape(1, 1)
    out = _layer1_dense(ahA, ahB, axA, axB, dr16, w1hi, w1lo, w1ba, w1bb,
                        W2, b2.reshape(1, _OUT), inv)
    return out.reshape(_OUT)


# speedup vs baseline: 1.1759x; 1.0301x over previous
import functools

import jax
import jax.numpy as jnp
from jax import lax
from jax.experimental import pallas as pl
from jax.experimental.pallas import tpu as pltpu
from jax.experimental.pallas import tpu_sc as plsc

_N = 10000
_E = 320000
_D = 128
_H = 256
_OUT = 128

_NT = 16
_NC = 2
_C = 80
_K0 = 12
_C1 = 80
_K1 = 4
_SBR1 = 50
_HALFW = 72
_DEGW = 8
_NP = 10240
_NPT = _NP // _NT

_mesh = plsc.VectorSubcoreMesh(core_axis_name="c", subcore_axis_name="s")
_f32 = jnp.float32


@functools.partial(
    pl.kernel,
    out_type=[jax.ShapeDtypeStruct((_NP, _DEGW), _f32),
              jax.ShapeDtypeStruct((_NP, _DEGW), _f32)],
    mesh=_mesh,
    scratch_types=[pltpu.VMEM((_C, _DEGW), _f32),
                   pltpu.VMEM((_SBR1, _C), jnp.int32),
                   pltpu.VMEM_SHARED((_NP, _DEGW), _f32),
                   pltpu.SemaphoreType.DMA],
    compiler_params=pltpu.CompilerParams(use_tc_tiling_on_sc=False),
)
def _deg_kernel(send2d_hbm, recv2d_hbm, ones_hbm, zeros_hbm, ds_hbm, dr_hbm,
                ones_v, idx_v, acc, sem):
    c = lax.axis_index("c")
    s = lax.axis_index("s")
    pltpu.sync_copy(zeros_hbm, acc.at[pl.ds(s * _NPT, _NPT)])
    pltpu.sync_copy(ones_hbm, ones_v)
    plsc.subcore_barrier()

    rpt = (_E // _C) // _NT
    nsb = rpt // _SBR1

    def hist(idx_hbm, out_hbm):
        def superblock(sb, carry):
            r0 = s * rpt + sb * _SBR1
            pltpu.sync_copy(idx_hbm.at[pl.ds(r0, _SBR1)], idx_v)
            adds = [pltpu.async_copy(ones_v, acc.at[idx_v.at[j]], sem,
                                     add=True)
                    for j in range(_SBR1)]
            for t in adds:
                t.wait()
            return carry
        lax.fori_loop(0, nsb, superblock, 0)
        plsc.subcore_barrier()
        pltpu.sync_copy(acc.at[pl.ds(s * _NPT, _NPT)],
                        out_hbm.at[pl.ds(s * _NPT, _NPT)])

    @pl.when(c == 0)
    def _():
        hist(send2d_hbm, ds_hbm)

    @pl.when(c == 1)
    def _():
        hist(recv2d_hbm, dr_hbm)


def _seg_body(table_hbm, out_hbm, send2d, recv2d, zeros_hbm,
              sidx, ridx, rows_v, acc, gsems, ssems, s, k, nsb, lag, csz, sbr):
    pltpu.sync_copy(zeros_hbm, acc.at[pl.ds(s * _NPT, _NPT)])
    plsc.subcore_barrier()
    row_base = s * (nsb * sbr)

    def superblock(sb, carry):
        r0 = row_base + sb * sbr
        pltpu.sync_copy(send2d.at[pl.ds(r0, sbr)], sidx)
        pltpu.sync_copy(recv2d.at[pl.ds(r0, sbr)], ridx)

        gdesc = [None] * sbr
        sdesc = [None] * sbr

        def fire_scatter(t):
            slot = t % k
            gdesc[t].wait()
            sdesc[t] = pltpu.async_copy(rows_v.at[pl.ds(slot * csz, csz)],
                                        acc.at[ridx.at[t]], ssems.at[slot],
                                        add=True)

        for j in range(sbr):
            slot = j % k
            if j >= k:
                sdesc[j - k].wait()
            gdesc[j] = pltpu.async_copy(table_hbm.at[sidx.at[j]],
                                        rows_v.at[pl.ds(slot * csz, csz)],
                                        gsems.at[slot])
            if j >= lag:
                fire_scatter(j - lag)
        for t in range(sbr - lag, sbr):
            fire_scatter(t)
        for t in range(sbr - k, sbr):
            sdesc[t].wait()
        return carry

    lax.fori_loop(0, nsb, superblock, 0)
    plsc.subcore_barrier()
    pltpu.sync_copy(acc.at[pl.ds(s * _NPT, _NPT)],
                    out_hbm.at[pl.ds(s * _NPT, _NPT)])


@functools.partial(
    pl.kernel,
    out_type=[jax.ShapeDtypeStruct((_NP, _HALFW), _f32),
              jax.ShapeDtypeStruct((_NP, _HALFW), _f32)],
    mesh=_mesh,
    scratch_types=[pltpu.VMEM((_SBR1, _C), jnp.int32),
                   pltpu.VMEM((_SBR1, _C), jnp.int32),
                   pltpu.VMEM((_K0 * _C, _HALFW), _f32),
                   pltpu.VMEM_SHARED((_NP, _HALFW), _f32),
                   pltpu.SemaphoreType.DMA((_K0,)),
                   pltpu.SemaphoreType.DMA((_K0,))],
    compiler_params=pltpu.CompilerParams(use_tc_tiling_on_sc=False),
)
def _l0_kernel(xnA_hbm, xnB_hbm, send2d_hbm, recv2d_hbm, zeros_hbm,
               axA_hbm, axB_hbm, sidx, ridx, rows_v, acc, gsem, ssem):
    c = lax.axis_index("c")
    s = lax.axis_index("s")
    rpt = (_E // _C) // _NT
    nsb = rpt // _SBR1

    @pl.when(c == 0)
    def _():
        _seg_body(xnA_hbm, axA_hbm, send2d_hbm, recv2d_hbm, zeros_hbm,
                  sidx, ridx, rows_v, acc, gsem, ssem, s, _K0, nsb,
                  8, _C, _SBR1)

    @pl.when(c == 1)
    def _():
        _seg_body(xnB_hbm, axB_hbm, send2d_hbm, recv2d_hbm, zeros_hbm,
                  sidx, ridx, rows_v, acc, gsem, ssem, s, _K0, nsb,
                  8, _C, _SBR1)


@functools.partial(
    pl.kernel,
    out_type=[jax.ShapeDtypeStruct((_NP, _D), _f32),
              jax.ShapeDtypeStruct((_NP, _D), _f32)],
    mesh=_mesh,
    scratch_types=[pltpu.VMEM((_SBR1, _C1), jnp.int32),
                   pltpu.VMEM((_SBR1, _C1), jnp.int32),
                   pltpu.VMEM((_K1 * _C1, _D), _f32),
                   pltpu.VMEM_SHARED((_NP, _D), _f32),
                   pltpu.SemaphoreType.DMA((_K1,)),
                   pltpu.SemaphoreType.DMA((_K1,))],
    compiler_params=pltpu.CompilerParams(use_tc_tiling_on_sc=False),
)
def _l1_kernel(hA_hbm, hB_hbm, send2d_hbm, recv2d_hbm, zeros_hbm,
               ahA_hbm, ahB_hbm, sidx, ridx, rows_v, acc, gsem, ssem):
    c = lax.axis_index("c")
    s = lax.axis_index("s")
    rpt = (_E // _C1) // _NT
    nsb = rpt // _SBR1

    @pl.when(c == 0)
    def _():
        _seg_body(hA_hbm, ahA_hbm, send2d_hbm, recv2d_hbm, zeros_hbm,
                  sidx, ridx, rows_v, acc, gsem, ssem, s, _K1, nsb,
                  3, _C1, _SBR1)

    @pl.when(c == 1)
    def _():
        _seg_body(hB_hbm, ahB_hbm, send2d_hbm, recv2d_hbm, zeros_hbm,
                  sidx, ridx, rows_v, acc, gsem, ssem, s, _K1, nsb,
                  3, _C1, _SBR1)


_BR = 2048
_BRP = 2000


def _prep_body(nodes_ref, ds_ref, outa_ref, outb_ref):
    ns = lax.rsqrt(jnp.maximum(ds_ref[:, 0], 1.0))
    xn = nodes_ref[...] * ns[:, None]
    rows = xn.shape[0]
    pad7 = jnp.zeros((rows, _HALFW - 65), _f32)
    pad8 = jnp.zeros((rows, _HALFW - 64), _f32)
    outa_ref[...] = jnp.concatenate([xn[:, :64], ns[:, None], pad7], axis=1)
    outb_ref[...] = jnp.concatenate([xn[:, 64:], pad8], axis=1)


def _prep(nodes, ds16):
    return pl.pallas_call(
        _prep_body,
        grid=(_N // _BRP,),
        in_specs=[pl.BlockSpec((_BRP, _D), lambda i: (i, 0)),
                  pl.BlockSpec((_BRP, _DEGW), lambda i: (i, 0))],
        out_specs=[pl.BlockSpec((_BRP, _HALFW), lambda i: (i, 0)),
                   pl.BlockSpec((_BRP, _HALFW), lambda i: (i, 0))],
        out_shape=[jax.ShapeDtypeStruct((_N, _HALFW), _f32),
                   jax.ShapeDtypeStruct((_N, _HALFW), _f32)],
    )(nodes, ds16)


def _layer0_dense_body(axA_ref, axB_ref, dr_ref, ds_ref, w0a_ref, w0b_ref,
                       outa_ref, outb_ref):
    t = (jnp.dot(axA_ref[...], w0a_ref[...], preferred_element_type=_f32)
         + jnp.dot(axB_ref[...], w0b_ref[...], preferred_element_type=_f32))
    nr = lax.rsqrt(jnp.maximum(dr_ref[:, 0], 1.0))
    ns = lax.rsqrt(jnp.maximum(ds_ref[:, 0], 1.0))
    h0n = jnp.maximum(t * nr[:, None], 0.0) * ns[:, None]
    outa_ref[...] = h0n[:, :_D]
    outb_ref[...] = h0n[:, _D:]


def _layer0_dense(axA, axB, dr16, ds16, w0a, w0b):
    return pl.pallas_call(
        _layer0_dense_body,
        grid=(_NP // _BR,),
        in_specs=[pl.BlockSpec((_BR, _HALFW), lambda i: (i, 0)),
                  pl.BlockSpec((_BR, _HALFW), lambda i: (i, 0)),
                  pl.BlockSpec((_BR, _DEGW), lambda i: (i, 0)),
                  pl.BlockSpec((_BR, _DEGW), lambda i: (i, 0)),
                  pl.BlockSpec((_HALFW, _H), lambda i: (0, 0)),
                  pl.BlockSpec((_HALFW, _H), lambda i: (0, 0))],
        out_specs=[pl.BlockSpec((_BR, _D), lambda i: (i, 0)),
                   pl.BlockSpec((_BR, _D), lambda i: (i, 0))],
        out_shape=[jax.ShapeDtypeStruct((_NP, _D), _f32),
                   jax.ShapeDtypeStruct((_NP, _D), _f32)],
    )(axA, axB, dr16, ds16, w0a, w0b)


def _layer1_dense_body(ahA_ref, ahB_ref, axA_ref, axB_ref, dr_ref,
                       w1hi_ref, w1lo_ref, w1ba_ref, w1bb_ref,
                       w2_ref, b2_ref, inv_ref, out_ref, acc_ref):
    i = pl.program_id(0)
    g = (jnp.dot(ahA_ref[...], w1hi_ref[...], preferred_element_type=_f32)
         + jnp.dot(ahB_ref[...], w1lo_ref[...], preferred_element_type=_f32)
         + jnp.dot(axA_ref[...], w1ba_ref[...], preferred_element_type=_f32)
         + jnp.dot(axB_ref[...], w1bb_ref[...], preferred_element_type=_f32))
    nr = lax.rsqrt(jnp.maximum(dr_ref[:, 0], 1.0))
    h1 = jnp.maximum(g * nr[:, None], 0.0)
    psum = jnp.sum(h1, axis=0, keepdims=True)
    acc_ref[...] = jnp.where(i == 0, psum, acc_ref[...] + psum)

    @pl.when(i == pl.num_programs(0) - 1)
    def _():
        pooled = acc_ref[...] * inv_ref[0, 0]
        out_ref[...] = (jnp.dot(pooled, w2_ref[...], preferred_element_type=_f32)
                        + b2_ref[...])


def _layer1_dense(ahA, ahB, axA, axB, dr16, w1hi, w1lo, w1ba, w1bb,
                  w2, b2, inv):
    return pl.pallas_call(
        _layer1_dense_body,
        grid=(_NP // _BR,),
        in_specs=[pl.BlockSpec((_BR, _D), lambda i: (i, 0)),
                  pl.BlockSpec((_BR, _D), lambda i: (i, 0)),
                  pl.BlockSpec((_BR, _HALFW), lambda i: (i, 0)),
                  pl.BlockSpec((_BR, _HALFW), lambda i: (i, 0)),
                  pl.BlockSpec((_BR, _DEGW), lambda i: (i, 0)),
                  pl.BlockSpec((_D, _H), lambda i: (0, 0)),
                  pl.BlockSpec((_D, _H), lambda i: (0, 0)),
                  pl.BlockSpec((_HALFW, _H), lambda i: (0, 0)),
                  pl.BlockSpec((_HALFW, _H), lambda i: (0, 0)),
                  pl.BlockSpec((_H, _OUT), lambda i: (0, 0)),
                  pl.BlockSpec((1, _OUT), lambda i: (0, 0)),
                  pl.BlockSpec((1, 1), lambda i: (0, 0))],
        out_specs=pl.BlockSpec((1, _OUT), lambda i: (0, 0)),
        out_shape=jax.ShapeDtypeStruct((1, _OUT), _f32),
        scratch_shapes=[pltpu.VMEM((1, _H), _f32)],
    )(ahA, ahB, axA, axB, dr16, w1hi, w1lo, w1ba, w1bb, w2, b2, inv)


def kernel(nodes, senders, receivers, n_node, W0, b0, W1, b1, W2, b2):
    ones16 = jnp.ones((_C, _DEGW), _f32)
    zeros_deg = jnp.zeros((_NPT, _DEGW), _f32)
    zeros_half = jnp.zeros((_NPT, _HALFW), _f32)
    zeros_d = jnp.zeros((_NPT, _D), _f32)
    send2d = senders.reshape(_E // _C, _C)
    recv2d = receivers.reshape(_E // _C, _C)
    send2d1 = senders.reshape(_E // _C1, _C1)
    recv2d1 = receivers.reshape(_E // _C1, _C1)

    ds16, dr16 = _deg_kernel(send2d, recv2d, ones16, zeros_deg)
    xnA, xnB = _prep(nodes, ds16)
    axA, axB = _l0_kernel(xnA, xnB, send2d, recv2d, zeros_half)

    pad7 = jnp.zeros((_HALFW - 65, _H), _f32)
    pad8 = jnp.zeros((_HALFW - 64, _H), _f32)
    w0a = jnp.concatenate([W0[:64], b0[None, :], pad7], axis=0)
    w0b = jnp.concatenate([W0[64:], pad8], axis=0)
    hA, hB = _layer0_dense(axA, axB, dr16, ds16, w0a, w0b)

    ahA, ahB = _l1_kernel(hA, hB, send2d1, recv2d1, zeros_d)

    w1hi = W1[:_D]
    w1lo = W1[_D:_H]
    w1ba = jnp.concatenate([W1[_H:_H + 64], b1[None, :], pad7], axis=0)
    w1bb = jnp.concatenate([W1[_H + 64:], pad8], axis=0)
    inv = (1.0 / jnp.maximum(n_node.astype(_f32), 1.0)).reshape(1, 1)
    out = _layer1_dense(ahA, ahB, axA, axB, dr16, w1hi, w1lo, w1ba, w1bb,
                        W2, b2.reshape(1, _OUT), inv)
    return out.reshape(_OUT)
